# Initial kernel scaffold; baseline (speedup 1.0000x reference)
import jax
import jax.numpy as jnp
from jax.experimental import pallas as pl


def kernel(z, pos, edge_index, params):
    raise NotImplementedError("write your pallas kernel here")


# speedup vs baseline: 41.2908x; 41.2908x over previous
import functools
import math

import jax
import jax.numpy as jnp
from jax import lax
from jax.experimental import pallas as pl
from jax.experimental.pallas import tpu as pltpu
from jax.experimental.pallas import tpu_sc as plsc

N_NODES = 10000
N_EDGES = 160000
HIDDEN = 32
NUM_RBF = 32
MAX_Z = 128
CUTOFF_UPPER = 4.5
EPS_LN = 1e-5

_NC = 2
_NS = 16
_NW = _NC * _NS
_EPAD = 163840
_CHUNK = _EPAD // _NW
_GROUPS = _CHUNK // 16

_BLK = 128
_NBLK = _EPAD // _BLK
_BPT = _NBLK // _NS
_TRASH = N_NODES
_AROWS = 10112
_RPT = _AROWS // _NS

_FB = 128


def _gather_body(posx, posy, posz, zt, srcp, dstp,
                 evxo, evyo, evzo, zso, zdo,
                 px, py, pz, zv, sv, dv, bevx, bevy, bevz, bzs, bzd):
    c = lax.axis_index("c")
    s = lax.axis_index("s")
    wid = s * _NC + c
    base = wid * _CHUNK
    pltpu.sync_copy(posx, px)
    pltpu.sync_copy(posy, py)
    pltpu.sync_copy(posz, pz)
    pltpu.sync_copy(zt, zv)
    pltpu.sync_copy(srcp.at[pl.ds(base, _CHUNK)], sv)
    pltpu.sync_copy(dstp.at[pl.ds(base, _CHUNK)], dv)

    def body(j, carry):
        e0 = j * 16
        isrc = sv[pl.ds(e0, 16)]
        idst = dv[pl.ds(e0, 16)]
        bevx[pl.ds(e0, 16)] = (plsc.load_gather(px, [isrc])
                               - plsc.load_gather(px, [idst]))
        bevy[pl.ds(e0, 16)] = (plsc.load_gather(py, [isrc])
                               - plsc.load_gather(py, [idst]))
        bevz[pl.ds(e0, 16)] = (plsc.load_gather(pz, [isrc])
                               - plsc.load_gather(pz, [idst]))
        bzs[pl.ds(e0, 16)] = plsc.load_gather(zv, [isrc])
        bzd[pl.ds(e0, 16)] = plsc.load_gather(zv, [idst])
        return carry

    lax.fori_loop(0, _GROUPS, body, jnp.int32(0))
    pltpu.sync_copy(bevx, evxo.at[pl.ds(base, _CHUNK)])
    pltpu.sync_copy(bevy, evyo.at[pl.ds(base, _CHUNK)])
    pltpu.sync_copy(bevz, evzo.at[pl.ds(base, _CHUNK)])
    pltpu.sync_copy(bzs, zso.at[pl.ds(base, _CHUNK)])
    pltpu.sync_copy(bzd, zdo.at[pl.ds(base, _CHUNK)])


@functools.cache
def _gather_kernel():
    return functools.partial(
        pl.kernel,
        mesh=plsc.VectorSubcoreMesh(core_axis_name="c", subcore_axis_name="s"),
        compiler_params=pltpu.CompilerParams(needs_layout_passes=False),
        out_type=(
            jax.ShapeDtypeStruct((_EPAD,), jnp.float32),
            jax.ShapeDtypeStruct((_EPAD,), jnp.float32),
            jax.ShapeDtypeStruct((_EPAD,), jnp.float32),
            jax.ShapeDtypeStruct((_EPAD,), jnp.int32),
            jax.ShapeDtypeStruct((_EPAD,), jnp.int32),
        ),
        scratch_types=[
            pltpu.VMEM((N_NODES,), jnp.float32),
            pltpu.VMEM((N_NODES,), jnp.float32),
            pltpu.VMEM((N_NODES,), jnp.float32),
            pltpu.VMEM((N_NODES,), jnp.int32),
            pltpu.VMEM((_CHUNK,), jnp.int32),
            pltpu.VMEM((_CHUNK,), jnp.int32),
            pltpu.VMEM((_CHUNK,), jnp.float32),
            pltpu.VMEM((_CHUNK,), jnp.float32),
            pltpu.VMEM((_CHUNK,), jnp.float32),
            pltpu.VMEM((_CHUNK,), jnp.int32),
            pltpu.VMEM((_CHUNK,), jnp.int32),
        ],
    )(_gather_body)


def _gather_call(posx, posy, posz, zt, srcp, dstp):
    return _gather_kernel()(posx, posy, posz, zt, srcp, dstp)


def _exp(x):
    x = jnp.clip(x, -87.0, 88.0)
    n = jnp.floor(x * 1.4426950408889634 + 0.5)
    z = x - n * 0.693359375
    z = z - n * (-2.12194440e-4)
    p = 1.0 + z * (1.0 + z * (0.5 + z * (
        (1.0 / 6.0) + z * ((1.0 / 24.0) + z * ((1.0 / 120.0) + z * (
            (1.0 / 720.0) + z * (1.0 / 5040.0)))))))
    e = lax.bitcast_convert_type((n.astype(jnp.int32) + 127) << 23,
                                 jnp.float32)
    return p * e


def _rsqrt(y):
    r = lax.rsqrt(y)
    return r * (1.5 - 0.5 * y * r * r)


def _recip(y):
    r = _rsqrt(y)
    return r * r


def _cutoff(d):
    x = d * (math.pi / 9.0)
    w = x * x
    c = 1.0 + w * (-0.5 + w * ((1.0 / 24.0) + w * (
        (-1.0 / 720.0) + w * ((1.0 / 40320.0) + w * (
            (-1.0 / 3628800.0) + w * (1.0 / 479001600.0))))))
    return jnp.where(d < CUTOFF_UPPER, c * c, 0.0)


def _silu(x):
    return x * _recip(1.0 + _exp(-x))


def _bdot(x, w):
    return jnp.dot(x.astype(jnp.bfloat16), w.astype(jnp.bfloat16),
                   preferred_element_type=jnp.float32)


_MSG_W = 512


def _msg_body(geom, zpair, emb, emb2_W, emb2_b, dp1_W, dp1_b, dp2_W, dp2_b,
              dp3_W, dp3_b, m0, m1, m2):
    f32 = jnp.float32
    evx = geom[:, 0:1]
    evy = geom[:, 1:2]
    evz = geom[:, 2:3]
    t = evx * evx + evy * evy + evz * evz
    t2 = t + 1e-12
    d = t2 * _rsqrt(t2)
    rdn = _recip(d + 1e-9)
    vx = evx * rdn
    vy = evy * rdn
    vz = evz * rdn
    cut = _cutoff(d)

    alpha = 5.0 / CUTOFF_UPPER
    start = math.exp(-CUTOFF_UPPER)
    beta = (2.0 / NUM_RBF * (1.0 - start)) ** -2
    ridx = lax.broadcasted_iota(jnp.int32, (1, NUM_RBF), 1).astype(f32)
    means = start + ridx * ((1.0 - start) / (NUM_RBF - 1))
    rbf = cut * _exp(-beta * (_exp(-alpha * d) - means) ** 2)

    zs = zpair[:, 0:1]
    zd = zpair[:, 1:2]
    ziota = lax.broadcasted_iota(jnp.int32, (_MSG_W, MAX_Z), 1)
    ohs = (zs == ziota).astype(f32)
    ohd = (zd == ziota).astype(f32)
    T1 = _bdot(emb[:], emb2_W[:HIDDEN, :])
    T2 = _bdot(emb[:], emb2_W[HIDDEN:, :])
    Zij = (jnp.dot(ohs, T1, preferred_element_type=f32,
                   precision=lax.Precision.HIGHEST)
           + jnp.dot(ohd, T2, preferred_element_type=f32,
                     precision=lax.Precision.HIGHEST) + emb2_b[:])
    C = cut * Zij

    c1 = (_bdot(rbf, dp1_W[:]) + dp1_b[:]) * C
    c2 = (_bdot(rbf, dp2_W[:]) + dp2_b[:]) * C
    c3 = (_bdot(rbf, dp3_W[:]) + dp3_b[:]) * C

    tn = vx * vx + vy * vy + vz * vz
    tn3 = tn * (1.0 / 3.0)
    zeros64 = jnp.zeros((_MSG_W, 2 * HIDDEN), f32)
    m0[:, :] = jnp.concatenate([c1, c2 * vx, c2 * vy, c2 * vz], axis=-1)
    m1[:, :] = jnp.concatenate(
        [c3 * (vx * vx - tn3), c3 * (vy * vy - tn3), c3 * (vz * vz - tn3),
         c3 * (vx * vy)], axis=-1)
    m2[:, :] = jnp.concatenate(
        [c3 * (vx * vz), c3 * (vy * vz), zeros64], axis=-1)


def _msg_call(geom, zpair, emb, emb2_W, emb2_b, dp1_W, dp1_b, dp2_W, dp2_b,
              dp3_W, dp3_b):
    grid = _EPAD // _MSG_W
    full = lambda i: (0, 0)
    mspec = pl.BlockSpec((_MSG_W, _FB), lambda i: (i, 0))
    mshape = jax.ShapeDtypeStruct((_EPAD, _FB), jnp.float32)
    return pl.pallas_call(
        _msg_body,
        grid=(grid,),
        in_specs=[
            pl.BlockSpec((_MSG_W, 3), lambda i: (i, 0)),
            pl.BlockSpec((_MSG_W, 2), lambda i: (i, 0)),
            pl.BlockSpec((MAX_Z, HIDDEN), full),
            pl.BlockSpec((2 * HIDDEN, HIDDEN), full),
            pl.BlockSpec((1, HIDDEN), full),
            pl.BlockSpec((NUM_RBF, HIDDEN), full),
            pl.BlockSpec((1, HIDDEN), full),
            pl.BlockSpec((NUM_RBF, HIDDEN), full),
            pl.BlockSpec((1, HIDDEN), full),
            pl.BlockSpec((NUM_RBF, HIDDEN), full),
            pl.BlockSpec((1, HIDDEN), full),
        ],
        out_specs=[mspec, mspec, mspec],
        out_shape=[mshape, mshape, mshape],
    )(geom, zpair, emb, emb2_W, emb2_b, dp1_W, dp1_b, dp2_W, dp2_b,
      dp3_W, dp3_b)


def _scatter_body(m0, m1, m2, srcb, zrows, o0, o1, o2a, o2b,
                  shared, idxv1, idxv2, mbuf):
    c = lax.axis_index("c")
    s = lax.axis_index("s")
    r0 = s * _RPT
    h = _NBLK // 2
    hpt = h // _NS
    pltpu.sync_copy(srcb.at[pl.ds(s * _BPT, _BPT)], idxv1)
    pltpu.sync_copy(srcb.at[pl.ds(c * h + s * hpt, hpt)], idxv2)

    def run_phase(msg, out, idxv, nb, blk0):
        pltpu.sync_copy(zrows.at[pl.ds(r0, _RPT)], shared.at[pl.ds(r0, _RPT)])
        plsc.subcore_barrier()

        def body(j, carry):
            b = blk0 + j
            pltpu.sync_copy(msg.at[pl.ds(b * _BLK, _BLK)], mbuf)
            pltpu.sync_copy(mbuf, shared.at[idxv.at[j]], add=True)
            return carry

        lax.fori_loop(0, nb, body, jnp.int32(0))
        plsc.subcore_barrier()
        pltpu.sync_copy(shared.at[pl.ds(r0, _RPT)], out.at[pl.ds(r0, _RPT)])
        plsc.subcore_barrier()

    @pl.when(c == 0)
    def _():
        run_phase(m0, o0, idxv1, _BPT, s * _BPT)
        run_phase(m2, o2a, idxv2, hpt, s * hpt)

    @pl.when(c == 1)
    def _():
        run_phase(m1, o1, idxv1, _BPT, s * _BPT)
        run_phase(m2, o2b, idxv2, hpt, h + s * hpt)


@functools.cache
def _scatter_kernel():
    oshape = jax.ShapeDtypeStruct((_AROWS, _FB), jnp.float32)
    return functools.partial(
        pl.kernel,
        mesh=plsc.VectorSubcoreMesh(core_axis_name="c", subcore_axis_name="s"),
        compiler_params=pltpu.CompilerParams(needs_layout_passes=False),
        out_type=(oshape, oshape, oshape, oshape),
        scratch_types=[
            pltpu.VMEM_SHARED((_AROWS, _FB), jnp.float32),
            pltpu.VMEM((_BPT, _BLK), jnp.int32),
            pltpu.VMEM((_NBLK // 2 // _NS, _BLK), jnp.int32),
            pltpu.VMEM((_BLK, _FB), jnp.float32),
        ],
    )(_scatter_body)


def _scatter_call(m0, m1, m2, srcb, zrows):
    return _scatter_kernel()(m0, m1, m2, srcb, zrows)


_NODE_W = 400


def _layer_norm(x, g, b):
    mu = jnp.mean(x, axis=-1, keepdims=True)
    xc = x - mu
    var = jnp.mean(xc * xc, axis=-1, keepdims=True)
    return xc * _rsqrt(var + EPS_LN) * g + b


def _node_body(h0, h1, h2a, h2b, lt0_W, lt1_W, lt2_W, ls0_W, ls0_b,
               ls1_W, ls1_b, in_g, in_b, on_g, on_b, lin_W, lin_b,
               ol1_W, ol1_b, ol2_W, ol2_b, y):
    f32 = jnp.float32
    H = HIDDEN
    x320 = jnp.concatenate([h0[:, :], h1[:, :], h2a[:, :] + h2b[:, :]],
                           axis=-1)
    g = lambda k: x320[:, k * H:(k + 1) * H]
    a = g(0)
    wx = g(1)
    wy = g(2)
    wz = g(3)
    sxx = g(4)
    syy = g(5)
    szz = g(6)
    sxy = g(7)
    sxz = g(8)
    syz = g(9)

    nrm = (3.0 * a * a
           + 2.0 * (wx * wx + wy * wy + wz * wz)
           + (sxx * sxx + syy * syy + szz * szz)
           + 2.0 * (sxy * sxy + sxz * sxz + syz * syz))
    nrm = _layer_norm(nrm, in_g[:], in_b[:])
    h1m = _silu(_bdot(nrm, ls0_W[:]) + ls0_b[:])
    h2m = _silu(_bdot(h1m, ls1_W[:]) + ls1_b[:])

    r = lax.broadcasted_iota(jnp.int32, (3 * H, H), 0)
    hcol = lax.broadcasted_iota(jnp.int32, (3 * H, H), 1)
    P0 = (r == 3 * hcol).astype(f32)
    P1 = (r == 3 * hcol + 1).astype(f32)
    P2 = (r == 3 * hcol + 2).astype(f32)
    g0 = jnp.dot(h2m, P0, preferred_element_type=f32, precision=lax.Precision.HIGHEST)
    g1 = jnp.dot(h2m, P1, preferred_element_type=f32, precision=lax.Precision.HIGHEST)
    g2 = jnp.dot(h2m, P2, preferred_element_type=f32, precision=lax.Precision.HIGHEST)

    dot = lambda x, W: _bdot(x, W[:])
    a2 = dot(a, lt0_W) * g0
    wx2 = dot(wx, lt1_W) * g1
    wy2 = dot(wy, lt1_W) * g1
    wz2 = dot(wz, lt1_W) * g1
    sxx2 = dot(sxx, lt2_W) * g2
    syy2 = dot(syy, lt2_W) * g2
    szz2 = dot(szz, lt2_W) * g2
    sxy2 = dot(sxy, lt2_W) * g2
    sxz2 = dot(sxz, lt2_W) * g2
    syz2 = dot(syz, lt2_W) * g2

    tnI = 3.0 * a2 * a2
    tnA = 2.0 * (wx2 * wx2 + wy2 * wy2 + wz2 * wz2)
    tnS = (sxx2 * sxx2 + syy2 * syy2 + szz2 * szz2
           + 2.0 * (sxy2 * sxy2 + sxz2 * sxz2 + syz2 * syz2))
    x = jnp.concatenate([tnI, tnA, tnS], axis=-1)
    x = _layer_norm(x, on_g[:], on_b[:])
    x = _silu(_bdot(x, lin_W[:]) + lin_b[:])
    x = _silu(_bdot(x, ol1_W[:]) + ol1_b[:])
    y[:, :] = _bdot(x, ol2_W[:]) + ol2_b[:]


def _node_call(h0, h1, h2a, h2b, p):
    H = HIDDEN
    grid = N_NODES // _NODE_W
    full = lambda i: (0, 0)
    row = lambda i: (i, 0)
    b2 = lambda v: v.reshape(1, -1)
    hspec = pl.BlockSpec((_NODE_W, _FB), row)
    return pl.pallas_call(
        _node_body,
        grid=(grid,),
        in_specs=[
            hspec, hspec, hspec, hspec,
            pl.BlockSpec((H, H), full),
            pl.BlockSpec((H, H), full),
            pl.BlockSpec((H, H), full),
            pl.BlockSpec((H, 2 * H), full),
            pl.BlockSpec((1, 2 * H), full),
            pl.BlockSpec((2 * H, 3 * H), full),
            pl.BlockSpec((1, 3 * H), full),
            pl.BlockSpec((1, H), full),
            pl.BlockSpec((1, H), full),
            pl.BlockSpec((1, 3 * H), full),
            pl.BlockSpec((1, 3 * H), full),
            pl.BlockSpec((3 * H, H), full),
            pl.BlockSpec((1, H), full),
            pl.BlockSpec((H, H // 2), full),
            pl.BlockSpec((1, H // 2), full),
            pl.BlockSpec((H // 2, 1), full),
            pl.BlockSpec((1, 1), full),
        ],
        out_specs=pl.BlockSpec((_NODE_W, 1), row),
        out_shape=jax.ShapeDtypeStruct((N_NODES, 1), jnp.float32),
    )(h0, h1, h2a, h2b, p['lt0_W'], p['lt1_W'], p['lt2_W'], p['ls0_W'],
      b2(p['ls0_b']), p['ls1_W'], b2(p['ls1_b']), b2(p['in_g']),
      b2(p['in_b']), b2(p['on_g']), b2(p['on_b']), p['lin_W'],
      b2(p['lin_b']), p['ol1_W'], b2(p['ol1_b']), p['ol2_W'],
      b2(p['ol2_b']))


def kernel(z, pos, edge_index, params):
    z = z.astype(jnp.int32)
    src = edge_index[0].astype(jnp.int32)
    dst = edge_index[1].astype(jnp.int32)
    pad = _EPAD - N_EDGES
    zpad = jnp.zeros((pad,), jnp.int32)
    srcp = jnp.concatenate([src, zpad])
    dstp = jnp.concatenate([dst, zpad])
    posx = pos[:, 0]
    posy = pos[:, 1]
    posz = pos[:, 2]

    evx, evy, evz, zs, zd = _gather_call(posx, posy, posz, z, srcp, dstp)
    geom = jnp.stack([evx, evy, evz], axis=1)
    zpair = jnp.stack([zs, zd], axis=1)

    p = params
    b2 = lambda v: v.reshape(1, -1)
    m0, m1, m2 = _msg_call(
        geom, zpair, p['emb'], p['emb2_W'], b2(p['emb2_b']),
        p['dp1_W'], b2(p['dp1_b']), p['dp2_W'], b2(p['dp2_b']),
        p['dp3_W'], b2(p['dp3_b']))

    srcb = jnp.concatenate(
        [src, jnp.full((pad,), _TRASH, jnp.int32)]).reshape(_NBLK, _BLK)
    zrows = jnp.zeros((_AROWS, _FB), jnp.float32)
    h0, h1, h2a, h2b = _scatter_call(m0, m1, m2, srcb, zrows)

    return _node_call(h0, h1, h2a, h2b, p)


# speedup vs baseline: 64.9962x; 1.5741x over previous
import functools
import math

import jax
import jax.numpy as jnp
from jax import lax
from jax.experimental import pallas as pl
from jax.experimental.pallas import tpu as pltpu
from jax.experimental.pallas import tpu_sc as plsc

N_NODES = 10000
N_EDGES = 160000
HIDDEN = 32
NUM_RBF = 32
MAX_Z = 128
CUTOFF_UPPER = 4.5
EPS_LN = 1e-5

_NC = 2
_NS = 16
_NW = _NC * _NS
_EPAD = 163840
_CHUNK = _EPAD // _NW
_GROUPS = _CHUNK // 16

_BLK = 128
_NBLK = _EPAD // _BLK
_BPT = _NBLK // _NS
_TRASH = N_NODES
_AROWS = 10112
_RPT = _AROWS // _NS

_FB = 128


def _gather_body(posx, posy, posz, zt, srcp, dstp,
                 evxo, evyo, evzo, zso, zdo,
                 px, py, pz, zv, sv, dv, bevx, bevy, bevz, bzs, bzd):
    c = lax.axis_index("c")
    s = lax.axis_index("s")
    wid = s * _NC + c
    base = wid * _CHUNK
    pltpu.sync_copy(posx, px)
    pltpu.sync_copy(posy, py)
    pltpu.sync_copy(posz, pz)
    pltpu.sync_copy(zt, zv)
    pltpu.sync_copy(srcp.at[pl.ds(base, _CHUNK)], sv)
    pltpu.sync_copy(dstp.at[pl.ds(base, _CHUNK)], dv)

    def body(j, carry):
        e0 = j * 16
        isrc = sv[pl.ds(e0, 16)]
        idst = dv[pl.ds(e0, 16)]
        bevx[pl.ds(e0, 16)] = (plsc.load_gather(px, [isrc])
                               - plsc.load_gather(px, [idst]))
        bevy[pl.ds(e0, 16)] = (plsc.load_gather(py, [isrc])
                               - plsc.load_gather(py, [idst]))
        bevz[pl.ds(e0, 16)] = (plsc.load_gather(pz, [isrc])
                               - plsc.load_gather(pz, [idst]))
        bzs[pl.ds(e0, 16)] = plsc.load_gather(zv, [isrc])
        bzd[pl.ds(e0, 16)] = plsc.load_gather(zv, [idst])
        return carry

    lax.fori_loop(0, _GROUPS, body, jnp.int32(0))
    pltpu.sync_copy(bevx, evxo.at[pl.ds(base, _CHUNK)])
    pltpu.sync_copy(bevy, evyo.at[pl.ds(base, _CHUNK)])
    pltpu.sync_copy(bevz, evzo.at[pl.ds(base, _CHUNK)])
    pltpu.sync_copy(bzs, zso.at[pl.ds(base, _CHUNK)])
    pltpu.sync_copy(bzd, zdo.at[pl.ds(base, _CHUNK)])


@functools.cache
def _gather_kernel():
    return functools.partial(
        pl.kernel,
        mesh=plsc.VectorSubcoreMesh(core_axis_name="c", subcore_axis_name="s"),
        compiler_params=pltpu.CompilerParams(needs_layout_passes=False),
        out_type=(
            jax.ShapeDtypeStruct((_EPAD,), jnp.float32),
            jax.ShapeDtypeStruct((_EPAD,), jnp.float32),
            jax.ShapeDtypeStruct((_EPAD,), jnp.float32),
            jax.ShapeDtypeStruct((_EPAD,), jnp.int32),
            jax.ShapeDtypeStruct((_EPAD,), jnp.int32),
        ),
        scratch_types=[
            pltpu.VMEM((N_NODES,), jnp.float32),
            pltpu.VMEM((N_NODES,), jnp.float32),
            pltpu.VMEM((N_NODES,), jnp.float32),
            pltpu.VMEM((N_NODES,), jnp.int32),
            pltpu.VMEM((_CHUNK,), jnp.int32),
            pltpu.VMEM((_CHUNK,), jnp.int32),
            pltpu.VMEM((_CHUNK,), jnp.float32),
            pltpu.VMEM((_CHUNK,), jnp.float32),
            pltpu.VMEM((_CHUNK,), jnp.float32),
            pltpu.VMEM((_CHUNK,), jnp.int32),
            pltpu.VMEM((_CHUNK,), jnp.int32),
        ],
    )(_gather_body)


def _gather_call(posx, posy, posz, zt, srcp, dstp):
    return _gather_kernel()(posx, posy, posz, zt, srcp, dstp)


def _exp(x):
    x = jnp.clip(x, -87.0, 88.0)
    n = jnp.floor(x * 1.4426950408889634 + 0.5)
    z = x - n * 0.693359375
    z = z - n * (-2.12194440e-4)
    p = 1.0 + z * (1.0 + z * (0.5 + z * (
        (1.0 / 6.0) + z * ((1.0 / 24.0) + z * ((1.0 / 120.0) + z * (
            (1.0 / 720.0) + z * (1.0 / 5040.0)))))))
    e = lax.bitcast_convert_type((n.astype(jnp.int32) + 127) << 23,
                                 jnp.float32)
    return p * e


def _rsqrt(y):
    r = lax.rsqrt(y)
    return r * (1.5 - 0.5 * y * r * r)


def _recip(y):
    r = _rsqrt(y)
    return r * r


def _cutoff(d):
    x = d * (math.pi / 9.0)
    w = x * x
    c = 1.0 + w * (-0.5 + w * ((1.0 / 24.0) + w * (
        (-1.0 / 720.0) + w * ((1.0 / 40320.0) + w * (
            (-1.0 / 3628800.0) + w * (1.0 / 479001600.0))))))
    return jnp.where(d < CUTOFF_UPPER, c * c, 0.0)


def _silu(x):
    return x * _recip(1.0 + _exp(-x))


def _bdot(x, w):
    return jnp.dot(x.astype(jnp.bfloat16), w.astype(jnp.bfloat16),
                   preferred_element_type=jnp.float32)


_MSG_W = 512


def _msg_body(gx, gy, gz, zsr, zdr, emb, embT1, embT2, emb2_b4, WA, bA,
              WB, bB, m0, m1, m2):
    f32 = jnp.float32
    evx = gx[0]
    evy = gy[0]
    evz = gz[0]
    t2 = evx * evx + evy * evy + evz * evz + 1e-12
    d = t2 * _rsqrt(t2)
    rdn = _recip(d + 1e-9)
    vx = evx * rdn
    vy = evy * rdn
    vz = evz * rdn
    cut = _cutoff(d)
    tn3 = (vx * vx + vy * vy + vz * vz) * (1.0 / 3.0)
    crd = cut * rdn

    alpha = 5.0 / CUTOFF_UPPER
    start = math.exp(-CUTOFF_UPPER)
    beta = (2.0 / NUM_RBF * (1.0 - start)) ** -2
    w = _exp(-alpha * d)

    rows = jnp.concatenate(
        [cut, evx * crd, evy * crd, evz * crd,
         cut * (vx * vx - tn3), cut * (vy * vy - tn3),
         cut * (vz * vz - tn3), cut * (vx * vy),
         cut * (vx * vz), cut * (vy * vz),
         w, zsr[0].astype(f32), zdr[0].astype(f32),
         jnp.zeros_like(cut), jnp.zeros_like(cut), jnp.zeros_like(cut)],
        axis=0)
    T = jnp.transpose(rows, (1, 0))

    w512 = T[:, 10:11]
    ridx = lax.broadcasted_iota(jnp.int32, (1, NUM_RBF), 1).astype(f32)
    means = start + ridx * ((1.0 - start) / (NUM_RBF - 1))
    inner = w512 - means
    rbf = T[:, 0:1] * _exp(-beta * inner * inner)

    hp = lax.Precision.HIGHEST
    zsf = T[:, 11:12]
    zdf = T[:, 12:13]
    ziota = lax.broadcasted_iota(jnp.int32, (_MSG_W, MAX_Z), 1).astype(f32)
    ohs = (zsf == ziota).astype(f32)
    ohd = (zdf == ziota).astype(f32)
    TA = _bdot(emb[:], embT1[:])
    TB = _bdot(emb[:], embT2[:])
    Zij4 = (jnp.dot(ohs, TA, preferred_element_type=f32, precision=hp)
            + jnp.dot(ohd, TB, preferred_element_type=f32, precision=hp)
            + emb2_b4[:])

    dpA = _bdot(rbf, WA[:]) + bA[:]
    dpB = _bdot(rbf, WB[:]) + bB[:]

    sel = T[:, 0:12]
    r12 = lax.broadcasted_iota(jnp.int32, (12, _FB), 0)
    g32 = lax.broadcasted_iota(jnp.int32, (12, _FB), 1) // HIDDEN
    G0 = (r12 == g32).astype(f32)
    G1 = (r12 == g32 + 4).astype(f32)
    G2 = jnp.logical_and(r12 == g32 + 8, g32 < 2).astype(f32)
    S0 = jnp.dot(sel, G0, preferred_element_type=f32, precision=hp)
    S1 = jnp.dot(sel, G1, preferred_element_type=f32, precision=hp)
    S2 = jnp.dot(sel, G2, preferred_element_type=f32, precision=hp)

    m0[:, :] = dpA * Zij4 * S0
    m1[:, :] = dpB * Zij4 * S1
    m2[:, :] = dpB * Zij4 * S2


def _msg_call(gx, gy, gz, zsr, zdr, emb, embT1, embT2, emb2_b4, WA, bA,
              WB, bB):
    grid = _EPAD // _MSG_W
    full = lambda i: (0, 0)
    pspec = pl.BlockSpec((1, 1, _MSG_W), lambda i: (i, 0, 0))
    mspec = pl.BlockSpec((_MSG_W, _FB), lambda i: (i, 0))
    mshape = jax.ShapeDtypeStruct((_EPAD, _FB), jnp.float32)
    return pl.pallas_call(
        _msg_body,
        grid=(grid,),
        in_specs=[
            pspec, pspec, pspec, pspec, pspec,
            pl.BlockSpec((MAX_Z, HIDDEN), full),
            pl.BlockSpec((HIDDEN, _FB), full),
            pl.BlockSpec((HIDDEN, _FB), full),
            pl.BlockSpec((1, _FB), full),
            pl.BlockSpec((NUM_RBF, _FB), full),
            pl.BlockSpec((1, _FB), full),
            pl.BlockSpec((NUM_RBF, _FB), full),
            pl.BlockSpec((1, _FB), full),
        ],
        out_specs=[mspec, mspec, mspec],
        out_shape=[mshape, mshape, mshape],
    )(gx, gy, gz, zsr, zdr, emb, embT1, embT2, emb2_b4, WA, bA, WB, bB)


def _scatter_body(m0, m1, m2, srcb, zrows, o0, o1, o2a, o2b,
                  shared, idxv1, idxv2, mbuf):
    c = lax.axis_index("c")
    s = lax.axis_index("s")
    r0 = s * _RPT
    h = _NBLK // 2
    hpt = h // _NS
    pltpu.sync_copy(srcb.at[pl.ds(s * _BPT, _BPT)], idxv1)
    pltpu.sync_copy(srcb.at[pl.ds(c * h + s * hpt, hpt)], idxv2)

    def run_phase(msg, out, idxv, nb, blk0):
        pltpu.sync_copy(zrows.at[pl.ds(r0, _RPT)], shared.at[pl.ds(r0, _RPT)])
        plsc.subcore_barrier()

        def body(j, carry):
            b = blk0 + j
            pltpu.sync_copy(msg.at[pl.ds(b * _BLK, _BLK)], mbuf)
            pltpu.sync_copy(mbuf, shared.at[idxv.at[j]], add=True)
            return carry

        lax.fori_loop(0, nb, body, jnp.int32(0))
        plsc.subcore_barrier()
        pltpu.sync_copy(shared.at[pl.ds(r0, _RPT)], out.at[pl.ds(r0, _RPT)])
        plsc.subcore_barrier()

    @pl.when(c == 0)
    def _():
        run_phase(m0, o0, idxv1, _BPT, s * _BPT)
        run_phase(m2, o2a, idxv2, hpt, s * hpt)

    @pl.when(c == 1)
    def _():
        run_phase(m1, o1, idxv1, _BPT, s * _BPT)
        run_phase(m2, o2b, idxv2, hpt, h + s * hpt)


@functools.cache
def _scatter_kernel():
    oshape = jax.ShapeDtypeStruct((_AROWS, _FB), jnp.float32)
    return functools.partial(
        pl.kernel,
        mesh=plsc.VectorSubcoreMesh(core_axis_name="c", subcore_axis_name="s"),
        compiler_params=pltpu.CompilerParams(needs_layout_passes=False),
        out_type=(oshape, oshape, oshape, oshape),
        scratch_types=[
            pltpu.VMEM_SHARED((_AROWS, _FB), jnp.float32),
            pltpu.VMEM((_BPT, _BLK), jnp.int32),
            pltpu.VMEM((_NBLK // 2 // _NS, _BLK), jnp.int32),
            pltpu.VMEM((_BLK, _FB), jnp.float32),
        ],
    )(_scatter_body)


def _scatter_call(m0, m1, m2, srcb, zrows):
    return _scatter_kernel()(m0, m1, m2, srcb, zrows)


_NODE_W = 400


def _layer_norm(x, g, b):
    mu = jnp.mean(x, axis=-1, keepdims=True)
    xc = x - mu
    var = jnp.mean(xc * xc, axis=-1, keepdims=True)
    return xc * _rsqrt(var + EPS_LN) * g + b


def _node_body(h0, h1, h2a, h2b, lt0_W, lt1_W, lt2_W, ls0_W, ls0_b,
               ls1_W, ls1_b, in_g, in_b, on_g, on_b, lin_W, lin_b,
               ol1_W, ol1_b, ol2_W, ol2_b, y):
    f32 = jnp.float32
    H = HIDDEN
    x320 = jnp.concatenate([h0[:, :], h1[:, :], h2a[:, :] + h2b[:, :]],
                           axis=-1)
    g = lambda k: x320[:, k * H:(k + 1) * H]
    a = g(0)
    wx = g(1)
    wy = g(2)
    wz = g(3)
    sxx = g(4)
    syy = g(5)
    szz = g(6)
    sxy = g(7)
    sxz = g(8)
    syz = g(9)

    nrm = (3.0 * a * a
           + 2.0 * (wx * wx + wy * wy + wz * wz)
           + (sxx * sxx + syy * syy + szz * szz)
           + 2.0 * (sxy * sxy + sxz * sxz + syz * syz))
    nrm = _layer_norm(nrm, in_g[:], in_b[:])
    h1m = _silu(_bdot(nrm, ls0_W[:]) + ls0_b[:])
    h2m = _silu(_bdot(h1m, ls1_W[:]) + ls1_b[:])

    r = lax.broadcasted_iota(jnp.int32, (3 * H, H), 0)
    hcol = lax.broadcasted_iota(jnp.int32, (3 * H, H), 1)
    P0 = (r == 3 * hcol).astype(f32)
    P1 = (r == 3 * hcol + 1).astype(f32)
    P2 = (r == 3 * hcol + 2).astype(f32)
    g0 = jnp.dot(h2m, P0, preferred_element_type=f32, precision=lax.Precision.HIGHEST)
    g1 = jnp.dot(h2m, P1, preferred_element_type=f32, precision=lax.Precision.HIGHEST)
    g2 = jnp.dot(h2m, P2, preferred_element_type=f32, precision=lax.Precision.HIGHEST)

    dot = lambda x, W: _bdot(x, W[:])
    a2 = dot(a, lt0_W) * g0
    wx2 = dot(wx, lt1_W) * g1
    wy2 = dot(wy, lt1_W) * g1
    wz2 = dot(wz, lt1_W) * g1
    sxx2 = dot(sxx, lt2_W) * g2
    syy2 = dot(syy, lt2_W) * g2
    szz2 = dot(szz, lt2_W) * g2
    sxy2 = dot(sxy, lt2_W) * g2
    sxz2 = dot(sxz, lt2_W) * g2
    syz2 = dot(syz, lt2_W) * g2

    tnI = 3.0 * a2 * a2
    tnA = 2.0 * (wx2 * wx2 + wy2 * wy2 + wz2 * wz2)
    tnS = (sxx2 * sxx2 + syy2 * syy2 + szz2 * szz2
           + 2.0 * (sxy2 * sxy2 + sxz2 * sxz2 + syz2 * syz2))
    x = jnp.concatenate([tnI, tnA, tnS], axis=-1)
    x = _layer_norm(x, on_g[:], on_b[:])
    x = _silu(_bdot(x, lin_W[:]) + lin_b[:])
    x = _silu(_bdot(x, ol1_W[:]) + ol1_b[:])
    y[:, :] = _bdot(x, ol2_W[:]) + ol2_b[:]


def _node_call(h0, h1, h2a, h2b, p):
    H = HIDDEN
    grid = N_NODES // _NODE_W
    full = lambda i: (0, 0)
    row = lambda i: (i, 0)
    b2 = lambda v: v.reshape(1, -1)
    hspec = pl.BlockSpec((_NODE_W, _FB), row)
    return pl.pallas_call(
        _node_body,
        grid=(grid,),
        in_specs=[
            hspec, hspec, hspec, hspec,
            pl.BlockSpec((H, H), full),
            pl.BlockSpec((H, H), full),
            pl.BlockSpec((H, H), full),
            pl.BlockSpec((H, 2 * H), full),
            pl.BlockSpec((1, 2 * H), full),
            pl.BlockSpec((2 * H, 3 * H), full),
            pl.BlockSpec((1, 3 * H), full),
            pl.BlockSpec((1, H), full),
            pl.BlockSpec((1, H), full),
            pl.BlockSpec((1, 3 * H), full),
            pl.BlockSpec((1, 3 * H), full),
            pl.BlockSpec((3 * H, H), full),
            pl.BlockSpec((1, H), full),
            pl.BlockSpec((H, H // 2), full),
            pl.BlockSpec((1, H // 2), full),
            pl.BlockSpec((H // 2, 1), full),
            pl.BlockSpec((1, 1), full),
        ],
        out_specs=pl.BlockSpec((_NODE_W, 1), row),
        out_shape=jax.ShapeDtypeStruct((N_NODES, 1), jnp.float32),
    )(h0, h1, h2a, h2b, p['lt0_W'], p['lt1_W'], p['lt2_W'], p['ls0_W'],
      b2(p['ls0_b']), p['ls1_W'], b2(p['ls1_b']), b2(p['in_g']),
      b2(p['in_b']), b2(p['on_g']), b2(p['on_b']), p['lin_W'],
      b2(p['lin_b']), p['ol1_W'], b2(p['ol1_b']), p['ol2_W'],
      b2(p['ol2_b']))


def kernel(z, pos, edge_index, params):
    z = z.astype(jnp.int32)
    src = edge_index[0].astype(jnp.int32)
    dst = edge_index[1].astype(jnp.int32)
    pad = _EPAD - N_EDGES
    zpad = jnp.zeros((pad,), jnp.int32)
    srcp = jnp.concatenate([src, zpad])
    dstp = jnp.concatenate([dst, zpad])
    posx = pos[:, 0]
    posy = pos[:, 1]
    posz = pos[:, 2]

    evx, evy, evz, zs, zd = _gather_call(posx, posy, posz, z, srcp, dstp)

    p = params
    H = HIDDEN
    cat = jnp.concatenate
    r3 = lambda a: a.reshape(_EPAD // _MSG_W, 1, _MSG_W)
    WA = cat([p['dp1_W'], p['dp2_W'], p['dp2_W'], p['dp2_W']], axis=1)
    bA = cat([p['dp1_b'], p['dp2_b'], p['dp2_b'], p['dp2_b']]).reshape(1, -1)
    WB = cat([p['dp3_W']] * 4, axis=1)
    bB = cat([p['dp3_b']] * 4).reshape(1, -1)
    e2a = cat([p['emb2_W'][:H]] * 4, axis=1)
    e2b = cat([p['emb2_W'][H:]] * 4, axis=1)
    b4 = cat([p['emb2_b']] * 4).reshape(1, -1)
    m0, m1, m2 = _msg_call(
        r3(evx), r3(evy), r3(evz), r3(zs), r3(zd),
        p['emb'], e2a, e2b, b4, WA, bA, WB, bB)

    srcb = jnp.concatenate(
        [src, jnp.full((pad,), _TRASH, jnp.int32)]).reshape(_NBLK, _BLK)
    zrows = jnp.zeros((_AROWS, _FB), jnp.float32)
    h0, h1, h2a, h2b = _scatter_call(m0, m1, m2, srcb, zrows)

    return _node_call(h0, h1, h2a, h2b, p)


# speedup vs baseline: 70.2921x; 1.0815x over previous
import functools
import math

import jax
import jax.numpy as jnp
from jax import lax
from jax.experimental import pallas as pl
from jax.experimental.pallas import tpu as pltpu
from jax.experimental.pallas import tpu_sc as plsc

N_NODES = 10000
N_EDGES = 160000
HIDDEN = 32
NUM_RBF = 32
MAX_Z = 128
CUTOFF_UPPER = 4.5
EPS_LN = 1e-5

_NC = 2
_NS = 16
_NW = _NC * _NS
_EPAD = 163840
_CHUNK = _EPAD // _NW
_GROUPS = _CHUNK // 16

_BLK = 128
_NBLK = _EPAD // _BLK
_BPT = _NBLK // _NS
_TRASH = N_NODES
_AROWS = 10112
_RPT = _AROWS // _NS

_FB = 128


def _gather_body(posx, posy, posz, zt, srcp, dstp,
                 evxo, evyo, evzo, zso, zdo,
                 px, py, pz, zv, sv, dv, bevx, bevy, bevz, bzs, bzd):
    c = lax.axis_index("c")
    s = lax.axis_index("s")
    wid = s * _NC + c
    base = wid * _CHUNK
    pltpu.sync_copy(posx, px)
    pltpu.sync_copy(posy, py)
    pltpu.sync_copy(posz, pz)
    pltpu.sync_copy(zt, zv)
    pltpu.sync_copy(srcp.at[pl.ds(base, _CHUNK)], sv)
    pltpu.sync_copy(dstp.at[pl.ds(base, _CHUNK)], dv)

    def body(j, carry):
        e0 = j * 16
        isrc = sv[pl.ds(e0, 16)]
        idst = dv[pl.ds(e0, 16)]
        bevx[pl.ds(e0, 16)] = (plsc.load_gather(px, [isrc])
                               - plsc.load_gather(px, [idst]))
        bevy[pl.ds(e0, 16)] = (plsc.load_gather(py, [isrc])
                               - plsc.load_gather(py, [idst]))
        bevz[pl.ds(e0, 16)] = (plsc.load_gather(pz, [isrc])
                               - plsc.load_gather(pz, [idst]))
        bzs[pl.ds(e0, 16)] = plsc.load_gather(zv, [isrc])
        bzd[pl.ds(e0, 16)] = plsc.load_gather(zv, [idst])
        return carry

    lax.fori_loop(0, _GROUPS, body, jnp.int32(0))
    pltpu.sync_copy(bevx, evxo.at[pl.ds(base, _CHUNK)])
    pltpu.sync_copy(bevy, evyo.at[pl.ds(base, _CHUNK)])
    pltpu.sync_copy(bevz, evzo.at[pl.ds(base, _CHUNK)])
    pltpu.sync_copy(bzs, zso.at[pl.ds(base, _CHUNK)])
    pltpu.sync_copy(bzd, zdo.at[pl.ds(base, _CHUNK)])


@functools.cache
def _gather_kernel():
    return functools.partial(
        pl.kernel,
        mesh=plsc.VectorSubcoreMesh(core_axis_name="c", subcore_axis_name="s"),
        compiler_params=pltpu.CompilerParams(needs_layout_passes=False),
        out_type=(
            jax.ShapeDtypeStruct((_EPAD,), jnp.float32),
            jax.ShapeDtypeStruct((_EPAD,), jnp.float32),
            jax.ShapeDtypeStruct((_EPAD,), jnp.float32),
            jax.ShapeDtypeStruct((_EPAD,), jnp.int32),
            jax.ShapeDtypeStruct((_EPAD,), jnp.int32),
        ),
        scratch_types=[
            pltpu.VMEM((N_NODES,), jnp.float32),
            pltpu.VMEM((N_NODES,), jnp.float32),
            pltpu.VMEM((N_NODES,), jnp.float32),
            pltpu.VMEM((N_NODES,), jnp.int32),
            pltpu.VMEM((_CHUNK,), jnp.int32),
            pltpu.VMEM((_CHUNK,), jnp.int32),
            pltpu.VMEM((_CHUNK,), jnp.float32),
            pltpu.VMEM((_CHUNK,), jnp.float32),
            pltpu.VMEM((_CHUNK,), jnp.float32),
            pltpu.VMEM((_CHUNK,), jnp.int32),
            pltpu.VMEM((_CHUNK,), jnp.int32),
        ],
    )(_gather_body)


def _gather_call(posx, posy, posz, zt, srcp, dstp):
    return _gather_kernel()(posx, posy, posz, zt, srcp, dstp)


def _exp(x):
    x = jnp.clip(x, -87.0, 88.0)
    n = jnp.floor(x * 1.4426950408889634 + 0.5)
    z = x - n * 0.693359375
    z = z - n * (-2.12194440e-4)
    p = 1.0 + z * (1.0 + z * (0.5 + z * (
        (1.0 / 6.0) + z * ((1.0 / 24.0) + z * ((1.0 / 120.0) + z * (
            (1.0 / 720.0) + z * (1.0 / 5040.0)))))))
    e = lax.bitcast_convert_type((n.astype(jnp.int32) + 127) << 23,
                                 jnp.float32)
    return p * e


def _rsqrt(y):
    r = lax.rsqrt(y)
    return r * (1.5 - 0.5 * y * r * r)


def _recip(y):
    r = _rsqrt(y)
    return r * r


def _cutoff(d):
    x = d * (math.pi / 9.0)
    w = x * x
    c = 1.0 + w * (-0.5 + w * ((1.0 / 24.0) + w * (
        (-1.0 / 720.0) + w * ((1.0 / 40320.0) + w * (
            (-1.0 / 3628800.0) + w * (1.0 / 479001600.0))))))
    return jnp.where(d < CUTOFF_UPPER, c * c, 0.0)


def _silu(x):
    return x * _recip(1.0 + _exp(-x))


def _bdot(x, w):
    return jnp.dot(x.astype(jnp.bfloat16), w.astype(jnp.bfloat16),
                   preferred_element_type=jnp.float32)


_MSG_W = 512


def _msg_body(gx, gy, gz, zsr, zdr, emb, embT1, embT2, emb2_b4, WA, bA,
              WB, bB, m0, m1, m2):
    f32 = jnp.float32
    evx = gx[0]
    evy = gy[0]
    evz = gz[0]
    t2 = evx * evx + evy * evy + evz * evz + 1e-12
    d = t2 * _rsqrt(t2)
    rdn = _recip(d + 1e-9)
    vx = evx * rdn
    vy = evy * rdn
    vz = evz * rdn
    cut = _cutoff(d)
    tn3 = (vx * vx + vy * vy + vz * vz) * (1.0 / 3.0)
    crd = cut * rdn

    alpha = 5.0 / CUTOFF_UPPER
    start = math.exp(-CUTOFF_UPPER)
    beta = (2.0 / NUM_RBF * (1.0 - start)) ** -2
    w = _exp(-alpha * d)

    rows = jnp.concatenate(
        [cut, evx * crd, evy * crd, evz * crd,
         cut * (vx * vx - tn3), cut * (vy * vy - tn3),
         cut * (vz * vz - tn3), cut * (vx * vy),
         cut * (vx * vz), cut * (vy * vz),
         w, zsr[0].astype(f32), zdr[0].astype(f32),
         jnp.zeros_like(cut), jnp.zeros_like(cut), jnp.zeros_like(cut)],
        axis=0)
    T = jnp.transpose(rows, (1, 0))

    w512 = T[:, 10:11]
    ridx = lax.broadcasted_iota(jnp.int32, (1, NUM_RBF), 1).astype(f32)
    means = start + ridx * ((1.0 - start) / (NUM_RBF - 1))
    inner = w512 - means
    rbf = T[:, 0:1] * _exp(-beta * inner * inner)

    hp = lax.Precision.HIGHEST
    zsf = T[:, 11:12]
    zdf = T[:, 12:13]
    ziota = lax.broadcasted_iota(jnp.int32, (_MSG_W, MAX_Z), 1).astype(f32)
    ohs = (zsf == ziota).astype(f32)
    ohd = (zdf == ziota).astype(f32)
    TA = _bdot(emb[:], embT1[:])
    TB = _bdot(emb[:], embT2[:])
    Zij4 = (jnp.dot(ohs, TA, preferred_element_type=f32, precision=hp)
            + jnp.dot(ohd, TB, preferred_element_type=f32, precision=hp)
            + emb2_b4[:])

    dpA = _bdot(rbf, WA[:]) + bA[:]
    dpB = _bdot(rbf, WB[:]) + bB[:]

    sel = T[:, 0:12]
    r12 = lax.broadcasted_iota(jnp.int32, (12, _FB), 0)
    g32 = lax.broadcasted_iota(jnp.int32, (12, _FB), 1) // HIDDEN
    G0 = (r12 == g32).astype(f32)
    G1 = (r12 == g32 + 4).astype(f32)
    G2 = jnp.logical_and(r12 == g32 + 8, g32 < 2).astype(f32)
    S0 = jnp.dot(sel, G0, preferred_element_type=f32, precision=hp)
    S1 = jnp.dot(sel, G1, preferred_element_type=f32, precision=hp)
    S2 = jnp.dot(sel, G2, preferred_element_type=f32, precision=hp)

    m0[:, :] = dpA * Zij4 * S0
    m1[:, :] = dpB * Zij4 * S1
    m2[:, :] = dpB * Zij4 * S2


def _msg_call(gx, gy, gz, zsr, zdr, emb, embT1, embT2, emb2_b4, WA, bA,
              WB, bB):
    grid = _EPAD // _MSG_W
    full = lambda i: (0, 0)
    pspec = pl.BlockSpec((1, 1, _MSG_W), lambda i: (i, 0, 0))
    mspec = pl.BlockSpec((_MSG_W, _FB), lambda i: (i, 0))
    mshape = jax.ShapeDtypeStruct((_EPAD, _FB), jnp.float32)
    return pl.pallas_call(
        _msg_body,
        grid=(grid,),
        in_specs=[
            pspec, pspec, pspec, pspec, pspec,
            pl.BlockSpec((MAX_Z, HIDDEN), full),
            pl.BlockSpec((HIDDEN, _FB), full),
            pl.BlockSpec((HIDDEN, _FB), full),
            pl.BlockSpec((1, _FB), full),
            pl.BlockSpec((NUM_RBF, _FB), full),
            pl.BlockSpec((1, _FB), full),
            pl.BlockSpec((NUM_RBF, _FB), full),
            pl.BlockSpec((1, _FB), full),
        ],
        out_specs=[mspec, mspec, mspec],
        out_shape=[mshape, mshape, mshape],
    )(gx, gy, gz, zsr, zdr, emb, embT1, embT2, emb2_b4, WA, bA, WB, bB)


def _scatter_body(m0, m1, m2, srcb, zrows, o0, o1, o2a, o2b,
                  shared, idxv1, idxv2, mbufA, mbufB, semA, semB):
    c = lax.axis_index("c")
    s = lax.axis_index("s")
    r0 = s * _RPT
    h = _NBLK // 2
    hpt = h // _NS
    pltpu.sync_copy(srcb.at[pl.ds(s * _BPT, _BPT)], idxv1)
    pltpu.sync_copy(srcb.at[pl.ds(c * h + s * hpt, hpt)], idxv2)

    def run_phase(msg, out, idxv, nb, blk0):
        pltpu.sync_copy(zrows.at[pl.ds(r0, _RPT)], shared.at[pl.ds(r0, _RPT)])
        plsc.subcore_barrier()

        def copy_in(b, buf, sem):
            return pltpu.make_async_copy(
                msg.at[pl.ds(b * _BLK, _BLK)], buf, sem)

        copy_in(blk0, mbufA, semA).start()

        def body(i, carry):
            b0 = blk0 + 2 * i
            copy_in(b0, mbufA, semA).wait()
            copy_in(b0 + 1, mbufB, semB).start()
            pltpu.sync_copy(mbufA, shared.at[idxv.at[2 * i]], add=True)
            copy_in(b0 + 1, mbufB, semB).wait()

            @pl.when(i < nb // 2 - 1)
            def _():
                copy_in(b0 + 2, mbufA, semA).start()

            pltpu.sync_copy(mbufB, shared.at[idxv.at[2 * i + 1]], add=True)
            return carry

        lax.fori_loop(0, nb // 2, body, jnp.int32(0))
        plsc.subcore_barrier()
        pltpu.sync_copy(shared.at[pl.ds(r0, _RPT)], out.at[pl.ds(r0, _RPT)])
        plsc.subcore_barrier()

    @pl.when(c == 0)
    def _():
        run_phase(m0, o0, idxv1, _BPT, s * _BPT)
        run_phase(m2, o2a, idxv2, hpt, s * hpt)

    @pl.when(c == 1)
    def _():
        run_phase(m1, o1, idxv1, _BPT, s * _BPT)
        run_phase(m2, o2b, idxv2, hpt, h + s * hpt)


@functools.cache
def _scatter_kernel():
    oshape = jax.ShapeDtypeStruct((_AROWS, _FB), jnp.float32)
    return functools.partial(
        pl.kernel,
        mesh=plsc.VectorSubcoreMesh(core_axis_name="c", subcore_axis_name="s"),
        compiler_params=pltpu.CompilerParams(needs_layout_passes=False),
        out_type=(oshape, oshape, oshape, oshape),
        scratch_types=[
            pltpu.VMEM_SHARED((_AROWS, _FB), jnp.float32),
            pltpu.VMEM((_BPT, _BLK), jnp.int32),
            pltpu.VMEM((_NBLK // 2 // _NS, _BLK), jnp.int32),
            pltpu.VMEM((_BLK, _FB), jnp.float32),
            pltpu.VMEM((_BLK, _FB), jnp.float32),
            pltpu.SemaphoreType.DMA,
            pltpu.SemaphoreType.DMA,
        ],
    )(_scatter_body)


def _scatter_call(m0, m1, m2, srcb, zrows):
    return _scatter_kernel()(m0, m1, m2, srcb, zrows)


_NODE_W = 400


def _layer_norm(x, g, b):
    mu = jnp.mean(x, axis=-1, keepdims=True)
    xc = x - mu
    var = jnp.mean(xc * xc, axis=-1, keepdims=True)
    return xc * _rsqrt(var + EPS_LN) * g + b


def _node_body(h0, h1, h2a, h2b, lt0_W, lt1_W, lt2_W, ls0_W, ls0_b,
               ls1_W, ls1_b, in_g, in_b, on_g, on_b, lin_W, lin_b,
               ol1_W, ol1_b, ol2_W, ol2_b, y):
    f32 = jnp.float32
    H = HIDDEN
    x320 = jnp.concatenate([h0[:, :], h1[:, :], h2a[:, :] + h2b[:, :]],
                           axis=-1)
    g = lambda k: x320[:, k * H:(k + 1) * H]
    a = g(0)
    wx = g(1)
    wy = g(2)
    wz = g(3)
    sxx = g(4)
    syy = g(5)
    szz = g(6)
    sxy = g(7)
    sxz = g(8)
    syz = g(9)

    nrm = (3.0 * a * a
           + 2.0 * (wx * wx + wy * wy + wz * wz)
           + (sxx * sxx + syy * syy + szz * szz)
           + 2.0 * (sxy * sxy + sxz * sxz + syz * syz))
    nrm = _layer_norm(nrm, in_g[:], in_b[:])
    h1m = _silu(_bdot(nrm, ls0_W[:]) + ls0_b[:])
    h2m = _silu(_bdot(h1m, ls1_W[:]) + ls1_b[:])

    r = lax.broadcasted_iota(jnp.int32, (3 * H, H), 0)
    hcol = lax.broadcasted_iota(jnp.int32, (3 * H, H), 1)
    P0 = (r == 3 * hcol).astype(f32)
    P1 = (r == 3 * hcol + 1).astype(f32)
    P2 = (r == 3 * hcol + 2).astype(f32)
    g0 = jnp.dot(h2m, P0, preferred_element_type=f32, precision=lax.Precision.HIGHEST)
    g1 = jnp.dot(h2m, P1, preferred_element_type=f32, precision=lax.Precision.HIGHEST)
    g2 = jnp.dot(h2m, P2, preferred_element_type=f32, precision=lax.Precision.HIGHEST)

    dot = lambda x, W: _bdot(x, W[:])
    a2 = dot(a, lt0_W) * g0
    wx2 = dot(wx, lt1_W) * g1
    wy2 = dot(wy, lt1_W) * g1
    wz2 = dot(wz, lt1_W) * g1
    sxx2 = dot(sxx, lt2_W) * g2
    syy2 = dot(syy, lt2_W) * g2
    szz2 = dot(szz, lt2_W) * g2
    sxy2 = dot(sxy, lt2_W) * g2
    sxz2 = dot(sxz, lt2_W) * g2
    syz2 = dot(syz, lt2_W) * g2

    tnI = 3.0 * a2 * a2
    tnA = 2.0 * (wx2 * wx2 + wy2 * wy2 + wz2 * wz2)
    tnS = (sxx2 * sxx2 + syy2 * syy2 + szz2 * szz2
           + 2.0 * (sxy2 * sxy2 + sxz2 * sxz2 + syz2 * syz2))
    x = jnp.concatenate([tnI, tnA, tnS], axis=-1)
    x = _layer_norm(x, on_g[:], on_b[:])
    x = _silu(_bdot(x, lin_W[:]) + lin_b[:])
    x = _silu(_bdot(x, ol1_W[:]) + ol1_b[:])
    y[:, :] = _bdot(x, ol2_W[:]) + ol2_b[:]


def _node_call(h0, h1, h2a, h2b, p):
    H = HIDDEN
    grid = N_NODES // _NODE_W
    full = lambda i: (0, 0)
    row = lambda i: (i, 0)
    b2 = lambda v: v.reshape(1, -1)
    hspec = pl.BlockSpec((_NODE_W, _FB), row)
    return pl.pallas_call(
        _node_body,
        grid=(grid,),
        in_specs=[
            hspec, hspec, hspec, hspec,
            pl.BlockSpec((H, H), full),
            pl.BlockSpec((H, H), full),
            pl.BlockSpec((H, H), full),
            pl.BlockSpec((H, 2 * H), full),
            pl.BlockSpec((1, 2 * H), full),
            pl.BlockSpec((2 * H, 3 * H), full),
            pl.BlockSpec((1, 3 * H), full),
            pl.BlockSpec((1, H), full),
            pl.BlockSpec((1, H), full),
            pl.BlockSpec((1, 3 * H), full),
            pl.BlockSpec((1, 3 * H), full),
            pl.BlockSpec((3 * H, H), full),
            pl.BlockSpec((1, H), full),
            pl.BlockSpec((H, H // 2), full),
            pl.BlockSpec((1, H // 2), full),
            pl.BlockSpec((H // 2, 1), full),
            pl.BlockSpec((1, 1), full),
        ],
        out_specs=pl.BlockSpec((_NODE_W, 1), row),
        out_shape=jax.ShapeDtypeStruct((N_NODES, 1), jnp.float32),
    )(h0, h1, h2a, h2b, p['lt0_W'], p['lt1_W'], p['lt2_W'], p['ls0_W'],
      b2(p['ls0_b']), p['ls1_W'], b2(p['ls1_b']), b2(p['in_g']),
      b2(p['in_b']), b2(p['on_g']), b2(p['on_b']), p['lin_W'],
      b2(p['lin_b']), p['ol1_W'], b2(p['ol1_b']), p['ol2_W'],
      b2(p['ol2_b']))


def kernel(z, pos, edge_index, params):
    z = z.astype(jnp.int32)
    src = edge_index[0].astype(jnp.int32)
    dst = edge_index[1].astype(jnp.int32)
    pad = _EPAD - N_EDGES
    zpad = jnp.zeros((pad,), jnp.int32)
    srcp = jnp.concatenate([src, zpad])
    dstp = jnp.concatenate([dst, zpad])
    posx = pos[:, 0]
    posy = pos[:, 1]
    posz = pos[:, 2]

    evx, evy, evz, zs, zd = _gather_call(posx, posy, posz, z, srcp, dstp)

    p = params
    H = HIDDEN
    cat = jnp.concatenate
    r3 = lambda a: a.reshape(_EPAD // _MSG_W, 1, _MSG_W)
    WA = cat([p['dp1_W'], p['dp2_W'], p['dp2_W'], p['dp2_W']], axis=1)
    bA = cat([p['dp1_b'], p['dp2_b'], p['dp2_b'], p['dp2_b']]).reshape(1, -1)
    WB = cat([p['dp3_W']] * 4, axis=1)
    bB = cat([p['dp3_b']] * 4).reshape(1, -1)
    e2a = cat([p['emb2_W'][:H]] * 4, axis=1)
    e2b = cat([p['emb2_W'][H:]] * 4, axis=1)
    b4 = cat([p['emb2_b']] * 4).reshape(1, -1)
    m0, m1, m2 = _msg_call(
        r3(evx), r3(evy), r3(evz), r3(zs), r3(zd),
        p['emb'], e2a, e2b, b4, WA, bA, WB, bB)

    srcb = jnp.concatenate(
        [src, jnp.full((pad,), _TRASH, jnp.int32)]).reshape(_NBLK, _BLK)
    zrows = jnp.zeros((_AROWS, _FB), jnp.float32)
    h0, h1, h2a, h2b = _scatter_call(m0, m1, m2, srcb, zrows)

    return _node_call(h0, h1, h2a, h2b, p)


# speedup vs baseline: 75.4345x; 1.0732x over previous
import functools
import math

import jax
import jax.numpy as jnp
from jax import lax
from jax.experimental import pallas as pl
from jax.experimental.pallas import tpu as pltpu
from jax.experimental.pallas import tpu_sc as plsc

N_NODES = 10000
N_EDGES = 160000
HIDDEN = 32
NUM_RBF = 32
MAX_Z = 128
CUTOFF_UPPER = 4.5
EPS_LN = 1e-5

_NC = 2
_NS = 16
_NW = _NC * _NS
_EPAD = 163840
_CHUNK = _EPAD // _NW
_GROUPS = _CHUNK // 16

_BLK = 128
_NBLK = _EPAD // _BLK
_BPT = _NBLK // _NS
_TRASH = N_NODES
_AROWS = 10112
_RPT = _AROWS // _NS

_FB = 128


def _gather_body(posx, posy, posz, zt, srcp, dstp,
                 evxo, evyo, evzo, zso, zdo,
                 px, py, pz, zv, sv, dv, bevx, bevy, bevz, bzs, bzd):
    c = lax.axis_index("c")
    s = lax.axis_index("s")
    wid = s * _NC + c
    base = wid * _CHUNK
    pltpu.sync_copy(posx, px)
    pltpu.sync_copy(posy, py)
    pltpu.sync_copy(posz, pz)
    pltpu.sync_copy(zt, zv)
    pltpu.sync_copy(srcp.at[pl.ds(base, _CHUNK)], sv)
    pltpu.sync_copy(dstp.at[pl.ds(base, _CHUNK)], dv)

    def body(j, carry):
        e0 = j * 16
        isrc = sv[pl.ds(e0, 16)]
        idst = dv[pl.ds(e0, 16)]
        bevx[pl.ds(e0, 16)] = (plsc.load_gather(px, [isrc])
                               - plsc.load_gather(px, [idst]))
        bevy[pl.ds(e0, 16)] = (plsc.load_gather(py, [isrc])
                               - plsc.load_gather(py, [idst]))
        bevz[pl.ds(e0, 16)] = (plsc.load_gather(pz, [isrc])
                               - plsc.load_gather(pz, [idst]))
        bzs[pl.ds(e0, 16)] = plsc.load_gather(zv, [isrc])
        bzd[pl.ds(e0, 16)] = plsc.load_gather(zv, [idst])
        return carry

    lax.fori_loop(0, _GROUPS, body, jnp.int32(0))
    pltpu.sync_copy(bevx, evxo.at[pl.ds(base, _CHUNK)])
    pltpu.sync_copy(bevy, evyo.at[pl.ds(base, _CHUNK)])
    pltpu.sync_copy(bevz, evzo.at[pl.ds(base, _CHUNK)])
    pltpu.sync_copy(bzs, zso.at[pl.ds(base, _CHUNK)])
    pltpu.sync_copy(bzd, zdo.at[pl.ds(base, _CHUNK)])


@functools.cache
def _gather_kernel():
    return functools.partial(
        pl.kernel,
        mesh=plsc.VectorSubcoreMesh(core_axis_name="c", subcore_axis_name="s"),
        compiler_params=pltpu.CompilerParams(needs_layout_passes=False),
        out_type=(
            jax.ShapeDtypeStruct((_EPAD,), jnp.float32),
            jax.ShapeDtypeStruct((_EPAD,), jnp.float32),
            jax.ShapeDtypeStruct((_EPAD,), jnp.float32),
            jax.ShapeDtypeStruct((_EPAD,), jnp.int32),
            jax.ShapeDtypeStruct((_EPAD,), jnp.int32),
        ),
        scratch_types=[
            pltpu.VMEM((N_NODES,), jnp.float32),
            pltpu.VMEM((N_NODES,), jnp.float32),
            pltpu.VMEM((N_NODES,), jnp.float32),
            pltpu.VMEM((N_NODES,), jnp.int32),
            pltpu.VMEM((_CHUNK,), jnp.int32),
            pltpu.VMEM((_CHUNK,), jnp.int32),
            pltpu.VMEM((_CHUNK,), jnp.float32),
            pltpu.VMEM((_CHUNK,), jnp.float32),
            pltpu.VMEM((_CHUNK,), jnp.float32),
            pltpu.VMEM((_CHUNK,), jnp.int32),
            pltpu.VMEM((_CHUNK,), jnp.int32),
        ],
    )(_gather_body)


def _gather_call(posx, posy, posz, zt, srcp, dstp):
    return _gather_kernel()(posx, posy, posz, zt, srcp, dstp)


def _exp(x):
    x = jnp.clip(x, -87.0, 88.0)
    n = jnp.floor(x * 1.4426950408889634 + 0.5)
    z = x - n * 0.693359375
    z = z - n * (-2.12194440e-4)
    p = 1.0 + z * (1.0 + z * (0.5 + z * (
        (1.0 / 6.0) + z * ((1.0 / 24.0) + z * ((1.0 / 120.0) + z * (
            (1.0 / 720.0) + z * (1.0 / 5040.0)))))))
    e = lax.bitcast_convert_type((n.astype(jnp.int32) + 127) << 23,
                                 jnp.float32)
    return p * e


def _rsqrt(y):
    r = lax.rsqrt(y)
    return r * (1.5 - 0.5 * y * r * r)


def _recip(y):
    r = _rsqrt(y)
    return r * r


def _cutoff(d):
    x = d * (math.pi / 9.0)
    w = x * x
    c = 1.0 + w * (-0.5 + w * ((1.0 / 24.0) + w * (
        (-1.0 / 720.0) + w * ((1.0 / 40320.0) + w * (
            (-1.0 / 3628800.0) + w * (1.0 / 479001600.0))))))
    return jnp.where(d < CUTOFF_UPPER, c * c, 0.0)


def _silu(x):
    return x * _recip(1.0 + _exp(-x))


def _bdot(x, w):
    return jnp.dot(x.astype(jnp.bfloat16), w.astype(jnp.bfloat16),
                   preferred_element_type=jnp.float32)


_MSG_W = 1024


def _msg_body(gx, gy, gz, zsr, zdr, emb, embT1, embT2, emb2_b4, WA, bA,
              WB, bB, m0, m1, m2):
    f32 = jnp.float32
    evx = gx[0]
    evy = gy[0]
    evz = gz[0]
    t2 = evx * evx + evy * evy + evz * evz + 1e-12
    d = t2 * _rsqrt(t2)
    rdn = _recip(d + 1e-9)
    vx = evx * rdn
    vy = evy * rdn
    vz = evz * rdn
    cut = _cutoff(d)
    tn3 = (vx * vx + vy * vy + vz * vz) * (1.0 / 3.0)
    crd = cut * rdn

    alpha = 5.0 / CUTOFF_UPPER
    start = math.exp(-CUTOFF_UPPER)
    beta = (2.0 / NUM_RBF * (1.0 - start)) ** -2
    w = _exp(-alpha * d)

    rows = jnp.concatenate(
        [cut, evx * crd, evy * crd, evz * crd,
         cut * (vx * vx - tn3), cut * (vy * vy - tn3),
         cut * (vz * vz - tn3), cut * (vx * vy),
         cut * (vx * vz), cut * (vy * vz),
         w, zsr[0].astype(f32), zdr[0].astype(f32),
         jnp.zeros_like(cut), jnp.zeros_like(cut), jnp.zeros_like(cut)],
        axis=0)
    T = jnp.transpose(rows, (1, 0))

    w512 = T[:, 10:11]
    ridx = lax.broadcasted_iota(jnp.int32, (1, NUM_RBF), 1).astype(f32)
    means = start + ridx * ((1.0 - start) / (NUM_RBF - 1))
    inner = w512 - means
    rbf = T[:, 0:1] * _exp(-beta * inner * inner)

    hp = lax.Precision.HIGHEST
    zsf = T[:, 11:12]
    zdf = T[:, 12:13]
    ziota = lax.broadcasted_iota(jnp.int32, (_MSG_W, MAX_Z), 1).astype(f32)
    ohs = (zsf == ziota).astype(f32)
    ohd = (zdf == ziota).astype(f32)
    TA = _bdot(emb[:], embT1[:])
    TB = _bdot(emb[:], embT2[:])
    Zij4 = (jnp.dot(ohs, TA, preferred_element_type=f32, precision=hp)
            + jnp.dot(ohd, TB, preferred_element_type=f32, precision=hp)
            + emb2_b4[:])

    dpA = _bdot(rbf, WA[:]) + bA[:]
    dpB = _bdot(rbf, WB[:]) + bB[:]

    sel = T[:, 0:12]
    r12 = lax.broadcasted_iota(jnp.int32, (12, _FB), 0)
    g32 = lax.broadcasted_iota(jnp.int32, (12, _FB), 1) // HIDDEN
    G0 = (r12 == g32).astype(f32)
    G1 = (r12 == g32 + 4).astype(f32)
    G2 = jnp.logical_and(r12 == g32 + 8, g32 < 2).astype(f32)
    S0 = jnp.dot(sel, G0, preferred_element_type=f32, precision=hp)
    S1 = jnp.dot(sel, G1, preferred_element_type=f32, precision=hp)
    S2 = jnp.dot(sel, G2, preferred_element_type=f32, precision=hp)

    m0[:, :] = dpA * Zij4 * S0
    m1[:, :] = dpB * Zij4 * S1
    m2[:, :] = dpB * Zij4 * S2


def _msg_call(gx, gy, gz, zsr, zdr, emb, embT1, embT2, emb2_b4, WA, bA,
              WB, bB):
    grid = _EPAD // _MSG_W
    full = lambda i: (0, 0)
    pspec = pl.BlockSpec((1, 1, _MSG_W), lambda i: (i, 0, 0))
    mspec = pl.BlockSpec((_MSG_W, _FB), lambda i: (i, 0))
    mshape = jax.ShapeDtypeStruct((_EPAD, _FB), jnp.float32)
    return pl.pallas_call(
        _msg_body,
        grid=(grid,),
        in_specs=[
            pspec, pspec, pspec, pspec, pspec,
            pl.BlockSpec((MAX_Z, HIDDEN), full),
            pl.BlockSpec((HIDDEN, _FB), full),
            pl.BlockSpec((HIDDEN, _FB), full),
            pl.BlockSpec((1, _FB), full),
            pl.BlockSpec((NUM_RBF, _FB), full),
            pl.BlockSpec((1, _FB), full),
            pl.BlockSpec((NUM_RBF, _FB), full),
            pl.BlockSpec((1, _FB), full),
        ],
        out_specs=[mspec, mspec, mspec],
        out_shape=[mshape, mshape, mshape],
    )(gx, gy, gz, zsr, zdr, emb, embT1, embT2, emb2_b4, WA, bA, WB, bB)


def _scatter_body(m0, m1, m2, srcb, zrows, o0, o1, o2a, o2b,
                  shared, idxv1, idxv2, mbufA, mbufB, semA, semB):
    c = lax.axis_index("c")
    s = lax.axis_index("s")
    r0 = s * _RPT
    h = _NBLK // 2
    hpt = h // _NS
    pltpu.sync_copy(srcb.at[pl.ds(s * _BPT, _BPT)], idxv1)
    pltpu.sync_copy(srcb.at[pl.ds(c * h + s * hpt, hpt)], idxv2)

    def run_phase(msg, out, idxv, nb, blk0):
        pltpu.sync_copy(zrows.at[pl.ds(r0, _RPT)], shared.at[pl.ds(r0, _RPT)])
        plsc.subcore_barrier()

        def copy_in(b, buf, sem):
            return pltpu.make_async_copy(
                msg.at[pl.ds(b * _BLK, _BLK)], buf, sem)

        copy_in(blk0, mbufA, semA).start()

        def body(i, carry):
            b0 = blk0 + 2 * i
            copy_in(b0, mbufA, semA).wait()
            copy_in(b0 + 1, mbufB, semB).start()
            pltpu.sync_copy(mbufA, shared.at[idxv.at[2 * i]], add=True)
            copy_in(b0 + 1, mbufB, semB).wait()

            @pl.when(i < nb // 2 - 1)
            def _():
                copy_in(b0 + 2, mbufA, semA).start()

            pltpu.sync_copy(mbufB, shared.at[idxv.at[2 * i + 1]], add=True)
            return carry

        lax.fori_loop(0, nb // 2, body, jnp.int32(0))
        plsc.subcore_barrier()
        pltpu.sync_copy(shared.at[pl.ds(r0, _RPT)], out.at[pl.ds(r0, _RPT)])
        plsc.subcore_barrier()

    @pl.when(c == 0)
    def _():
        run_phase(m0, o0, idxv1, _BPT, s * _BPT)
        run_phase(m2, o2a, idxv2, hpt, s * hpt)

    @pl.when(c == 1)
    def _():
        run_phase(m1, o1, idxv1, _BPT, s * _BPT)
        run_phase(m2, o2b, idxv2, hpt, h + s * hpt)


@functools.cache
def _scatter_kernel():
    oshape = jax.ShapeDtypeStruct((_AROWS, _FB), jnp.float32)
    return functools.partial(
        pl.kernel,
        mesh=plsc.VectorSubcoreMesh(core_axis_name="c", subcore_axis_name="s"),
        compiler_params=pltpu.CompilerParams(needs_layout_passes=False),
        out_type=(oshape, oshape, oshape, oshape),
        scratch_types=[
            pltpu.VMEM_SHARED((_AROWS, _FB), jnp.float32),
            pltpu.VMEM((_BPT, _BLK), jnp.int32),
            pltpu.VMEM((_NBLK // 2 // _NS, _BLK), jnp.int32),
            pltpu.VMEM((_BLK, _FB), jnp.float32),
            pltpu.VMEM((_BLK, _FB), jnp.float32),
            pltpu.SemaphoreType.DMA,
            pltpu.SemaphoreType.DMA,
        ],
    )(_scatter_body)


def _scatter_call(m0, m1, m2, srcb, zrows):
    return _scatter_kernel()(m0, m1, m2, srcb, zrows)


_NODE_W = 1000


def _layer_norm(x, g, b):
    mu = jnp.mean(x, axis=-1, keepdims=True)
    xc = x - mu
    var = jnp.mean(xc * xc, axis=-1, keepdims=True)
    return xc * _rsqrt(var + EPS_LN) * g + b


def _node_body(h0, h1, h2a, h2b, lt0_W, lt1_W, lt2_W, ls0_W, ls0_b,
               ls1_W, ls1_b, in_g, in_b, on_g, on_b, lin_W, lin_b,
               ol1_W, ol1_b, ol2_W, ol2_b, y):
    f32 = jnp.float32
    H = HIDDEN
    x320 = jnp.concatenate([h0[:, :], h1[:, :], h2a[:, :] + h2b[:, :]],
                           axis=-1)
    g = lambda k: x320[:, k * H:(k + 1) * H]
    a = g(0)
    wx = g(1)
    wy = g(2)
    wz = g(3)
    sxx = g(4)
    syy = g(5)
    szz = g(6)
    sxy = g(7)
    sxz = g(8)
    syz = g(9)

    nrm = (3.0 * a * a
           + 2.0 * (wx * wx + wy * wy + wz * wz)
           + (sxx * sxx + syy * syy + szz * szz)
           + 2.0 * (sxy * sxy + sxz * sxz + syz * syz))
    nrm = _layer_norm(nrm, in_g[:], in_b[:])
    h1m = _silu(_bdot(nrm, ls0_W[:]) + ls0_b[:])
    h2m = _silu(_bdot(h1m, ls1_W[:]) + ls1_b[:])

    r = lax.broadcasted_iota(jnp.int32, (3 * H, H), 0)
    hcol = lax.broadcasted_iota(jnp.int32, (3 * H, H), 1)
    P0 = (r == 3 * hcol).astype(f32)
    P1 = (r == 3 * hcol + 1).astype(f32)
    P2 = (r == 3 * hcol + 2).astype(f32)
    g0 = jnp.dot(h2m, P0, preferred_element_type=f32, precision=lax.Precision.HIGHEST)
    g1 = jnp.dot(h2m, P1, preferred_element_type=f32, precision=lax.Precision.HIGHEST)
    g2 = jnp.dot(h2m, P2, preferred_element_type=f32, precision=lax.Precision.HIGHEST)

    dot = lambda x, W: _bdot(x, W[:])
    a2 = dot(a, lt0_W) * g0
    wx2 = dot(wx, lt1_W) * g1
    wy2 = dot(wy, lt1_W) * g1
    wz2 = dot(wz, lt1_W) * g1
    sxx2 = dot(sxx, lt2_W) * g2
    syy2 = dot(syy, lt2_W) * g2
    szz2 = dot(szz, lt2_W) * g2
    sxy2 = dot(sxy, lt2_W) * g2
    sxz2 = dot(sxz, lt2_W) * g2
    syz2 = dot(syz, lt2_W) * g2

    tnI = 3.0 * a2 * a2
    tnA = 2.0 * (wx2 * wx2 + wy2 * wy2 + wz2 * wz2)
    tnS = (sxx2 * sxx2 + syy2 * syy2 + szz2 * szz2
           + 2.0 * (sxy2 * sxy2 + sxz2 * sxz2 + syz2 * syz2))
    x = jnp.concatenate([tnI, tnA, tnS], axis=-1)
    x = _layer_norm(x, on_g[:], on_b[:])
    x = _silu(_bdot(x, lin_W[:]) + lin_b[:])
    x = _silu(_bdot(x, ol1_W[:]) + ol1_b[:])
    y[:, :] = _bdot(x, ol2_W[:]) + ol2_b[:]


def _node_call(h0, h1, h2a, h2b, p):
    H = HIDDEN
    grid = N_NODES // _NODE_W
    full = lambda i: (0, 0)
    row = lambda i: (i, 0)
    b2 = lambda v: v.reshape(1, -1)
    hspec = pl.BlockSpec((_NODE_W, _FB), row)
    return pl.pallas_call(
        _node_body,
        grid=(grid,),
        in_specs=[
            hspec, hspec, hspec, hspec,
            pl.BlockSpec((H, H), full),
            pl.BlockSpec((H, H), full),
            pl.BlockSpec((H, H), full),
            pl.BlockSpec((H, 2 * H), full),
            pl.BlockSpec((1, 2 * H), full),
            pl.BlockSpec((2 * H, 3 * H), full),
            pl.BlockSpec((1, 3 * H), full),
            pl.BlockSpec((1, H), full),
            pl.BlockSpec((1, H), full),
            pl.BlockSpec((1, 3 * H), full),
            pl.BlockSpec((1, 3 * H), full),
            pl.BlockSpec((3 * H, H), full),
            pl.BlockSpec((1, H), full),
            pl.BlockSpec((H, H // 2), full),
            pl.BlockSpec((1, H // 2), full),
            pl.BlockSpec((H // 2, 1), full),
            pl.BlockSpec((1, 1), full),
        ],
        out_specs=pl.BlockSpec((_NODE_W, 1), row),
        out_shape=jax.ShapeDtypeStruct((N_NODES, 1), jnp.float32),
    )(h0, h1, h2a, h2b, p['lt0_W'], p['lt1_W'], p['lt2_W'], p['ls0_W'],
      b2(p['ls0_b']), p['ls1_W'], b2(p['ls1_b']), b2(p['in_g']),
      b2(p['in_b']), b2(p['on_g']), b2(p['on_b']), p['lin_W'],
      b2(p['lin_b']), p['ol1_W'], b2(p['ol1_b']), p['ol2_W'],
      b2(p['ol2_b']))


def kernel(z, pos, edge_index, params):
    z = z.astype(jnp.int32)
    src = edge_index[0].astype(jnp.int32)
    dst = edge_index[1].astype(jnp.int32)
    pad = _EPAD - N_EDGES
    zpad = jnp.zeros((pad,), jnp.int32)
    srcp = jnp.concatenate([src, zpad])
    dstp = jnp.concatenate([dst, zpad])
    posx = pos[:, 0]
    posy = pos[:, 1]
    posz = pos[:, 2]

    evx, evy, evz, zs, zd = _gather_call(posx, posy, posz, z, srcp, dstp)

    p = params
    H = HIDDEN
    cat = jnp.concatenate
    r3 = lambda a: a.reshape(_EPAD // _MSG_W, 1, _MSG_W)
    WA = cat([p['dp1_W'], p['dp2_W'], p['dp2_W'], p['dp2_W']], axis=1)
    bA = cat([p['dp1_b'], p['dp2_b'], p['dp2_b'], p['dp2_b']]).reshape(1, -1)
    WB = cat([p['dp3_W']] * 4, axis=1)
    bB = cat([p['dp3_b']] * 4).reshape(1, -1)
    e2a = cat([p['emb2_W'][:H]] * 4, axis=1)
    e2b = cat([p['emb2_W'][H:]] * 4, axis=1)
    b4 = cat([p['emb2_b']] * 4).reshape(1, -1)
    m0, m1, m2 = _msg_call(
        r3(evx), r3(evy), r3(evz), r3(zs), r3(zd),
        p['emb'], e2a, e2b, b4, WA, bA, WB, bB)

    srcb = jnp.concatenate(
        [src, jnp.full((pad,), _TRASH, jnp.int32)]).reshape(_NBLK, _BLK)
    zrows = jnp.zeros((_AROWS, _FB), jnp.float32)
    h0, h1, h2a, h2b = _scatter_call(m0, m1, m2, srcb, zrows)

    return _node_call(h0, h1, h2a, h2b, p)


# speedup vs baseline: 77.8464x; 1.0320x over previous
import functools
import math

import jax
import jax.numpy as jnp
from jax import lax
from jax.experimental import pallas as pl
from jax.experimental.pallas import tpu as pltpu
from jax.experimental.pallas import tpu_sc as plsc

N_NODES = 10000
N_EDGES = 160000
HIDDEN = 32
NUM_RBF = 32
MAX_Z = 128
CUTOFF_UPPER = 4.5
EPS_LN = 1e-5

_NC = 2
_NS = 16
_NW = _NC * _NS
_EPAD = 163840
_CHUNK = _EPAD // _NW
_GROUPS = _CHUNK // 16

_BLK = 128
_NBLK = _EPAD // _BLK
_BPT = _NBLK // _NS
_TRASH = N_NODES
_AROWS = 10112
_RPT = _AROWS // _NS

_FB = 128


def _gather_body(posx, posy, posz, zt, srcp, dstp,
                 evxo, evyo, evzo, zso, zdo,
                 px, py, pz, zv, sv, dv, bevx, bevy, bevz, bzs, bzd):
    c = lax.axis_index("c")
    s = lax.axis_index("s")
    wid = s * _NC + c
    base = wid * _CHUNK
    pltpu.sync_copy(posx, px)
    pltpu.sync_copy(posy, py)
    pltpu.sync_copy(posz, pz)
    pltpu.sync_copy(zt, zv)
    pltpu.sync_copy(srcp.at[pl.ds(base, _CHUNK)], sv)
    pltpu.sync_copy(dstp.at[pl.ds(base, _CHUNK)], dv)

    def body(j, carry):
        e0 = j * 16
        isrc = sv[pl.ds(e0, 16)]
        idst = dv[pl.ds(e0, 16)]
        bevx[pl.ds(e0, 16)] = (plsc.load_gather(px, [isrc])
                               - plsc.load_gather(px, [idst]))
        bevy[pl.ds(e0, 16)] = (plsc.load_gather(py, [isrc])
                               - plsc.load_gather(py, [idst]))
        bevz[pl.ds(e0, 16)] = (plsc.load_gather(pz, [isrc])
                               - plsc.load_gather(pz, [idst]))
        bzs[pl.ds(e0, 16)] = plsc.load_gather(zv, [isrc])
        bzd[pl.ds(e0, 16)] = plsc.load_gather(zv, [idst])
        return carry

    lax.fori_loop(0, _GROUPS, body, jnp.int32(0))
    pltpu.sync_copy(bevx, evxo.at[pl.ds(base, _CHUNK)])
    pltpu.sync_copy(bevy, evyo.at[pl.ds(base, _CHUNK)])
    pltpu.sync_copy(bevz, evzo.at[pl.ds(base, _CHUNK)])
    pltpu.sync_copy(bzs, zso.at[pl.ds(base, _CHUNK)])
    pltpu.sync_copy(bzd, zdo.at[pl.ds(base, _CHUNK)])


@functools.cache
def _gather_kernel():
    return functools.partial(
        pl.kernel,
        mesh=plsc.VectorSubcoreMesh(core_axis_name="c", subcore_axis_name="s"),
        compiler_params=pltpu.CompilerParams(needs_layout_passes=False),
        out_type=(
            jax.ShapeDtypeStruct((_EPAD,), jnp.float32),
            jax.ShapeDtypeStruct((_EPAD,), jnp.float32),
            jax.ShapeDtypeStruct((_EPAD,), jnp.float32),
            jax.ShapeDtypeStruct((_EPAD,), jnp.int32),
            jax.ShapeDtypeStruct((_EPAD,), jnp.int32),
        ),
        scratch_types=[
            pltpu.VMEM((N_NODES,), jnp.float32),
            pltpu.VMEM((N_NODES,), jnp.float32),
            pltpu.VMEM((N_NODES,), jnp.float32),
            pltpu.VMEM((N_NODES,), jnp.int32),
            pltpu.VMEM((_CHUNK,), jnp.int32),
            pltpu.VMEM((_CHUNK,), jnp.int32),
            pltpu.VMEM((_CHUNK,), jnp.float32),
            pltpu.VMEM((_CHUNK,), jnp.float32),
            pltpu.VMEM((_CHUNK,), jnp.float32),
            pltpu.VMEM((_CHUNK,), jnp.int32),
            pltpu.VMEM((_CHUNK,), jnp.int32),
        ],
    )(_gather_body)


def _gather_call(posx, posy, posz, zt, srcp, dstp):
    return _gather_kernel()(posx, posy, posz, zt, srcp, dstp)


def _exp(x):
    x = jnp.clip(x, -87.0, 88.0)
    n = jnp.floor(x * 1.4426950408889634 + 0.5)
    z = x - n * 0.693359375
    z = z - n * (-2.12194440e-4)
    p = 1.0 + z * (1.0 + z * (0.5 + z * (
        (1.0 / 6.0) + z * ((1.0 / 24.0) + z * ((1.0 / 120.0) + z * (
            (1.0 / 720.0) + z * (1.0 / 5040.0)))))))
    e = lax.bitcast_convert_type((n.astype(jnp.int32) + 127) << 23,
                                 jnp.float32)
    return p * e


def _rsqrt(y):
    r = lax.rsqrt(y)
    return r * (1.5 - 0.5 * y * r * r)


def _recip(y):
    r = _rsqrt(y)
    return r * r


def _cutoff(d):
    x = d * (math.pi / 9.0)
    w = x * x
    c = 1.0 + w * (-0.5 + w * ((1.0 / 24.0) + w * (
        (-1.0 / 720.0) + w * ((1.0 / 40320.0) + w * (
            (-1.0 / 3628800.0) + w * (1.0 / 479001600.0))))))
    return jnp.where(d < CUTOFF_UPPER, c * c, 0.0)


def _silu(x):
    return x * _recip(1.0 + _exp(-x))


def _bdot(x, w):
    return jnp.dot(x.astype(jnp.bfloat16), w.astype(jnp.bfloat16),
                   preferred_element_type=jnp.float32)


_MSG_W = 2048


def _msg_body(gx, gy, gz, zsr, zdr, emb, embT1, embT2, emb2_b4, WA, bA,
              WB, bB, m0, m1, m2):
    f32 = jnp.float32
    evx = gx[0]
    evy = gy[0]
    evz = gz[0]
    t2 = evx * evx + evy * evy + evz * evz + 1e-12
    d = t2 * _rsqrt(t2)
    rdn = _recip(d + 1e-9)
    vx = evx * rdn
    vy = evy * rdn
    vz = evz * rdn
    cut = _cutoff(d)
    tn3 = (vx * vx + vy * vy + vz * vz) * (1.0 / 3.0)
    crd = cut * rdn

    alpha = 5.0 / CUTOFF_UPPER
    start = math.exp(-CUTOFF_UPPER)
    beta = (2.0 / NUM_RBF * (1.0 - start)) ** -2
    w = _exp(-alpha * d)

    rows = jnp.concatenate(
        [cut, evx * crd, evy * crd, evz * crd,
         cut * (vx * vx - tn3), cut * (vy * vy - tn3),
         cut * (vz * vz - tn3), cut * (vx * vy),
         cut * (vx * vz), cut * (vy * vz),
         w, zsr[0].astype(f32), zdr[0].astype(f32),
         jnp.zeros_like(cut), jnp.zeros_like(cut), jnp.zeros_like(cut)],
        axis=0)
    T = jnp.transpose(rows, (1, 0))

    w512 = T[:, 10:11]
    ridx = lax.broadcasted_iota(jnp.int32, (1, NUM_RBF), 1).astype(f32)
    means = start + ridx * ((1.0 - start) / (NUM_RBF - 1))
    inner = w512 - means
    rbf = T[:, 0:1] * _exp(-beta * inner * inner)

    hp = lax.Precision.HIGHEST
    zsf = T[:, 11:12]
    zdf = T[:, 12:13]
    ziota = lax.broadcasted_iota(jnp.int32, (_MSG_W, MAX_Z), 1).astype(f32)
    ohs = (zsf == ziota).astype(f32)
    ohd = (zdf == ziota).astype(f32)
    TA = _bdot(emb[:], embT1[:])
    TB = _bdot(emb[:], embT2[:])
    Zij4 = (jnp.dot(ohs, TA, preferred_element_type=f32, precision=hp)
            + jnp.dot(ohd, TB, preferred_element_type=f32, precision=hp)
            + emb2_b4[:])

    dpA = _bdot(rbf, WA[:]) + bA[:]
    dpB = _bdot(rbf, WB[:]) + bB[:]

    sel = T[:, 0:12]
    r12 = lax.broadcasted_iota(jnp.int32, (12, _FB), 0)
    g32 = lax.broadcasted_iota(jnp.int32, (12, _FB), 1) // HIDDEN
    G0 = (r12 == g32).astype(f32)
    G1 = (r12 == g32 + 4).astype(f32)
    G2 = jnp.logical_and(r12 == g32 + 8, g32 < 2).astype(f32)
    S0 = jnp.dot(sel, G0, preferred_element_type=f32, precision=hp)
    S1 = jnp.dot(sel, G1, preferred_element_type=f32, precision=hp)
    S2 = jnp.dot(sel, G2, preferred_element_type=f32, precision=hp)

    m0[:, :] = dpA * Zij4 * S0
    m1[:, :] = dpB * Zij4 * S1
    m2[:, :] = dpB * Zij4 * S2


def _msg_call(gx, gy, gz, zsr, zdr, emb, embT1, embT2, emb2_b4, WA, bA,
              WB, bB):
    grid = _EPAD // _MSG_W
    full = lambda i: (0, 0)
    pspec = pl.BlockSpec((1, 1, _MSG_W), lambda i: (i, 0, 0))
    mspec = pl.BlockSpec((_MSG_W, _FB), lambda i: (i, 0))
    mshape = jax.ShapeDtypeStruct((_EPAD, _FB), jnp.float32)
    return pl.pallas_call(
        _msg_body,
        grid=(grid,),
        in_specs=[
            pspec, pspec, pspec, pspec, pspec,
            pl.BlockSpec((MAX_Z, HIDDEN), full),
            pl.BlockSpec((HIDDEN, _FB), full),
            pl.BlockSpec((HIDDEN, _FB), full),
            pl.BlockSpec((1, _FB), full),
            pl.BlockSpec((NUM_RBF, _FB), full),
            pl.BlockSpec((1, _FB), full),
            pl.BlockSpec((NUM_RBF, _FB), full),
            pl.BlockSpec((1, _FB), full),
        ],
        out_specs=[mspec, mspec, mspec],
        out_shape=[mshape, mshape, mshape],
    )(gx, gy, gz, zsr, zdr, emb, embT1, embT2, emb2_b4, WA, bA, WB, bB)


def _scatter_body(m0, m1, m2, srcb, zrows, o0, o1, o2a, o2b,
                  shared, idxv1, idxv2, mbufA, mbufB, semA, semB):
    c = lax.axis_index("c")
    s = lax.axis_index("s")
    r0 = s * _RPT
    h = _NBLK // 2
    hpt = h // _NS
    pltpu.sync_copy(srcb.at[pl.ds(s * _BPT, _BPT)], idxv1)
    pltpu.sync_copy(srcb.at[pl.ds(c * h + s * hpt, hpt)], idxv2)

    def run_phase(msg, out, idxv, nb, blk0):
        pltpu.sync_copy(zrows.at[pl.ds(r0, _RPT)], shared.at[pl.ds(r0, _RPT)])
        plsc.subcore_barrier()

        def copy_in(b, buf, sem):
            return pltpu.make_async_copy(
                msg.at[pl.ds(b * _BLK, _BLK)], buf, sem)

        copy_in(blk0, mbufA, semA).start()

        def body(i, carry):
            b0 = blk0 + 2 * i
            copy_in(b0, mbufA, semA).wait()
            copy_in(b0 + 1, mbufB, semB).start()
            pltpu.sync_copy(mbufA, shared.at[idxv.at[2 * i]], add=True)
            copy_in(b0 + 1, mbufB, semB).wait()

            @pl.when(i < nb // 2 - 1)
            def _():
                copy_in(b0 + 2, mbufA, semA).start()

            pltpu.sync_copy(mbufB, shared.at[idxv.at[2 * i + 1]], add=True)
            return carry

        lax.fori_loop(0, nb // 2, body, jnp.int32(0))
        plsc.subcore_barrier()
        pltpu.sync_copy(shared.at[pl.ds(r0, _RPT)], out.at[pl.ds(r0, _RPT)])
        plsc.subcore_barrier()

    @pl.when(c == 0)
    def _():
        run_phase(m0, o0, idxv1, _BPT, s * _BPT)
        run_phase(m2, o2a, idxv2, hpt, s * hpt)

    @pl.when(c == 1)
    def _():
        run_phase(m1, o1, idxv1, _BPT, s * _BPT)
        run_phase(m2, o2b, idxv2, hpt, h + s * hpt)


@functools.cache
def _scatter_kernel():
    oshape = jax.ShapeDtypeStruct((_AROWS, _FB), jnp.float32)
    return functools.partial(
        pl.kernel,
        mesh=plsc.VectorSubcoreMesh(core_axis_name="c", subcore_axis_name="s"),
        compiler_params=pltpu.CompilerParams(needs_layout_passes=False),
        out_type=(oshape, oshape, oshape, oshape),
        scratch_types=[
            pltpu.VMEM_SHARED((_AROWS, _FB), jnp.float32),
            pltpu.VMEM((_BPT, _BLK), jnp.int32),
            pltpu.VMEM((_NBLK // 2 // _NS, _BLK), jnp.int32),
            pltpu.VMEM((_BLK, _FB), jnp.float32),
            pltpu.VMEM((_BLK, _FB), jnp.float32),
            pltpu.SemaphoreType.DMA,
            pltpu.SemaphoreType.DMA,
        ],
    )(_scatter_body)


def _scatter_call(m0, m1, m2, srcb, zrows):
    return _scatter_kernel()(m0, m1, m2, srcb, zrows)


_NODE_W = 1000


def _layer_norm(x, g, b):
    mu = jnp.mean(x, axis=-1, keepdims=True)
    xc = x - mu
    var = jnp.mean(xc * xc, axis=-1, keepdims=True)
    return xc * _rsqrt(var + EPS_LN) * g + b


def _node_body(h0, h1, h2a, h2b, lt0_W, lt1_W, lt2_W, ls0_W, ls0_b,
               ls1_W, ls1_b, in_g, in_b, on_g, on_b, lin_W, lin_b,
               ol1_W, ol1_b, ol2_W, ol2_b, y):
    f32 = jnp.float32
    H = HIDDEN
    x320 = jnp.concatenate([h0[:, :], h1[:, :], h2a[:, :] + h2b[:, :]],
                           axis=-1)
    g = lambda k: x320[:, k * H:(k + 1) * H]
    a = g(0)
    wx = g(1)
    wy = g(2)
    wz = g(3)
    sxx = g(4)
    syy = g(5)
    szz = g(6)
    sxy = g(7)
    sxz = g(8)
    syz = g(9)

    nrm = (3.0 * a * a
           + 2.0 * (wx * wx + wy * wy + wz * wz)
           + (sxx * sxx + syy * syy + szz * szz)
           + 2.0 * (sxy * sxy + sxz * sxz + syz * syz))
    nrm = _layer_norm(nrm, in_g[:], in_b[:])
    h1m = _silu(_bdot(nrm, ls0_W[:]) + ls0_b[:])
    h2m = _silu(_bdot(h1m, ls1_W[:]) + ls1_b[:])

    r = lax.broadcasted_iota(jnp.int32, (3 * H, H), 0)
    hcol = lax.broadcasted_iota(jnp.int32, (3 * H, H), 1)
    P0 = (r == 3 * hcol).astype(f32)
    P1 = (r == 3 * hcol + 1).astype(f32)
    P2 = (r == 3 * hcol + 2).astype(f32)
    g0 = jnp.dot(h2m, P0, preferred_element_type=f32, precision=lax.Precision.HIGHEST)
    g1 = jnp.dot(h2m, P1, preferred_element_type=f32, precision=lax.Precision.HIGHEST)
    g2 = jnp.dot(h2m, P2, preferred_element_type=f32, precision=lax.Precision.HIGHEST)

    dot = lambda x, W: _bdot(x, W[:])
    a2 = dot(a, lt0_W) * g0
    wx2 = dot(wx, lt1_W) * g1
    wy2 = dot(wy, lt1_W) * g1
    wz2 = dot(wz, lt1_W) * g1
    sxx2 = dot(sxx, lt2_W) * g2
    syy2 = dot(syy, lt2_W) * g2
    szz2 = dot(szz, lt2_W) * g2
    sxy2 = dot(sxy, lt2_W) * g2
    sxz2 = dot(sxz, lt2_W) * g2
    syz2 = dot(syz, lt2_W) * g2

    tnI = 3.0 * a2 * a2
    tnA = 2.0 * (wx2 * wx2 + wy2 * wy2 + wz2 * wz2)
    tnS = (sxx2 * sxx2 + syy2 * syy2 + szz2 * szz2
           + 2.0 * (sxy2 * sxy2 + sxz2 * sxz2 + syz2 * syz2))
    x = jnp.concatenate([tnI, tnA, tnS], axis=-1)
    x = _layer_norm(x, on_g[:], on_b[:])
    x = _silu(_bdot(x, lin_W[:]) + lin_b[:])
    x = _silu(_bdot(x, ol1_W[:]) + ol1_b[:])
    y[:, :] = _bdot(x, ol2_W[:]) + ol2_b[:]


def _node_call(h0, h1, h2a, h2b, p):
    H = HIDDEN
    grid = N_NODES // _NODE_W
    full = lambda i: (0, 0)
    row = lambda i: (i, 0)
    b2 = lambda v: v.reshape(1, -1)
    hspec = pl.BlockSpec((_NODE_W, _FB), row)
    return pl.pallas_call(
        _node_body,
        grid=(grid,),
        in_specs=[
            hspec, hspec, hspec, hspec,
            pl.BlockSpec((H, H), full),
            pl.BlockSpec((H, H), full),
            pl.BlockSpec((H, H), full),
            pl.BlockSpec((H, 2 * H), full),
            pl.BlockSpec((1, 2 * H), full),
            pl.BlockSpec((2 * H, 3 * H), full),
            pl.BlockSpec((1, 3 * H), full),
            pl.BlockSpec((1, H), full),
            pl.BlockSpec((1, H), full),
            pl.BlockSpec((1, 3 * H), full),
            pl.BlockSpec((1, 3 * H), full),
            pl.BlockSpec((3 * H, H), full),
            pl.BlockSpec((1, H), full),
            pl.BlockSpec((H, H // 2), full),
            pl.BlockSpec((1, H // 2), full),
            pl.BlockSpec((H // 2, 1), full),
            pl.BlockSpec((1, 1), full),
        ],
        out_specs=pl.BlockSpec((_NODE_W, 1), row),
        out_shape=jax.ShapeDtypeStruct((N_NODES, 1), jnp.float32),
    )(h0, h1, h2a, h2b, p['lt0_W'], p['lt1_W'], p['lt2_W'], p['ls0_W'],
      b2(p['ls0_b']), p['ls1_W'], b2(p['ls1_b']), b2(p['in_g']),
      b2(p['in_b']), b2(p['on_g']), b2(p['on_b']), p['lin_W'],
      b2(p['lin_b']), p['ol1_W'], b2(p['ol1_b']), p['ol2_W'],
      b2(p['ol2_b']))


def kernel(z, pos, edge_index, params):
    z = z.astype(jnp.int32)
    src = edge_index[0].astype(jnp.int32)
    dst = edge_index[1].astype(jnp.int32)
    pad = _EPAD - N_EDGES
    zpad = jnp.zeros((pad,), jnp.int32)
    srcp = jnp.concatenate([src, zpad])
    dstp = jnp.concatenate([dst, zpad])
    posx = pos[:, 0]
    posy = pos[:, 1]
    posz = pos[:, 2]

    evx, evy, evz, zs, zd = _gather_call(posx, posy, posz, z, srcp, dstp)

    p = params
    H = HIDDEN
    cat = jnp.concatenate
    r3 = lambda a: a.reshape(_EPAD // _MSG_W, 1, _MSG_W)
    WA = cat([p['dp1_W'], p['dp2_W'], p['dp2_W'], p['dp2_W']], axis=1)
    bA = cat([p['dp1_b'], p['dp2_b'], p['dp2_b'], p['dp2_b']]).reshape(1, -1)
    WB = cat([p['dp3_W']] * 4, axis=1)
    bB = cat([p['dp3_b']] * 4).reshape(1, -1)
    e2a = cat([p['emb2_W'][:H]] * 4, axis=1)
    e2b = cat([p['emb2_W'][H:]] * 4, axis=1)
    b4 = cat([p['emb2_b']] * 4).reshape(1, -1)
    m0, m1, m2 = _msg_call(
        r3(evx), r3(evy), r3(evz), r3(zs), r3(zd),
        p['emb'], e2a, e2b, b4, WA, bA, WB, bB)

    srcb = jnp.concatenate(
        [src, jnp.full((pad,), _TRASH, jnp.int32)]).reshape(_NBLK, _BLK)
    zrows = jnp.zeros((_AROWS, _FB), jnp.float32)
    h0, h1, h2a, h2b = _scatter_call(m0, m1, m2, srcb, zrows)

    return _node_call(h0, h1, h2a, h2b, p)
